# baseline (speedup 1.0000x reference)
import functools

import jax
import jax.numpy as jnp
from jax import lax
from jax.experimental import pallas as pl
from jax.experimental.pallas import tpu as pltpu
from jax.experimental.pallas import tpu_sc as plsc

N = 10000
E = 320000
D = 128
K = 16
H = 512
C = 10
G = 64

NUM_CORES = 2
NUM_SUBCORES = 16
NW = NUM_CORES * NUM_SUBCORES
EPW = E // NW
STEP = 128
EPW_P = 10240
NSTEP = EPW_P // STEP
GROUP = 16
NGROUP = NSTEP // GROUP
SINK = N
ROWS_A = 624
TAIL = N - NUM_SUBCORES * ROWS_A
TAIL0 = NUM_SUBCORES * ROWS_A


@functools.partial(jax.jit, static_argnames=("fc",))
def _sc_spmm(table, srcr, dstr, zeros, fc):
    mesh = plsc.VectorSubcoreMesh(core_axis_name="c", subcore_axis_name="s")

    @functools.partial(
        pl.kernel,
        mesh=mesh,
        out_type=jax.ShapeDtypeStruct((NUM_CORES, N, fc), jnp.float32),
        scratch_types=[
            pltpu.VMEM((GROUP, STEP), jnp.int32),
            pltpu.VMEM((GROUP, STEP), jnp.int32),
            pltpu.VMEM((2, STEP, fc), jnp.float32),
            pltpu.VMEM_SHARED((N + 256, fc), jnp.float32),
            pltpu.SemaphoreType.DMA,
            pltpu.SemaphoreType.DMA,
        ],
    )
    def k(table_hbm, src_hbm, dst_hbm, zeros_hbm, out_hbm,
          sidx, didx, rows, acc_ref, sem0, sem1):
        c = lax.axis_index("c")
        s = lax.axis_index("s")
        wid = c * NUM_SUBCORES + s
        r0 = s * ROWS_A
        pltpu.sync_copy(zeros_hbm.at[pl.ds(0, ROWS_A)], acc_ref.at[pl.ds(r0, ROWS_A)])

        @pl.when(s == NUM_SUBCORES - 1)
        def _ztail():
            pltpu.sync_copy(zeros_hbm.at[pl.ds(0, TAIL)],
                            acc_ref.at[pl.ds(TAIL0, TAIL)])

        plsc.subcore_barrier()

        def group(g, carry):
            pltpu.sync_copy(src_hbm.at[wid, g], sidx)
            pltpu.sync_copy(dst_hbm.at[wid, g], didx)
            pltpu.async_copy(table_hbm.at[sidx.at[0]], rows.at[0], sem0)

            def pair(p, carry2):
                j = 2 * p
                pltpu.async_copy(table_hbm.at[sidx.at[j + 1]], rows.at[1], sem1)
                pltpu.make_async_copy(table_hbm.at[sidx.at[j]],
                                      rows.at[0], sem0).wait()
                pltpu.sync_copy(rows.at[0], acc_ref.at[didx.at[j]], add=True)

                @pl.when(j + 2 < GROUP)
                def _next():
                    pltpu.async_copy(table_hbm.at[sidx.at[j + 2]],
                                     rows.at[0], sem0)

                pltpu.make_async_copy(table_hbm.at[sidx.at[j + 1]],
                                      rows.at[1], sem1).wait()
                pltpu.sync_copy(rows.at[1], acc_ref.at[didx.at[j + 1]], add=True)
                return carry2

            lax.fori_loop(0, GROUP // 2, pair, 0)
            return carry

        lax.fori_loop(0, NGROUP, group, 0)
        plsc.subcore_barrier()
        pltpu.sync_copy(acc_ref.at[pl.ds(r0, ROWS_A)],
                        out_hbm.at[c, pl.ds(r0, ROWS_A)])

        @pl.when(s == NUM_SUBCORES - 1)
        def _otail():
            pltpu.sync_copy(acc_ref.at[pl.ds(TAIL0, TAIL)],
                            out_hbm.at[c, pl.ds(TAIL0, TAIL)])

    return k(table, srcr, dstr, zeros)


RBLK = 400
NBLK = N // RBLK


def _mlp_body(nch, residual, *refs):
    i = 0
    h_refs = refs[i:i + nch]; i += nch
    lap_ref = refs[i]; i += 1
    a_refs = refs[i:i + nch]; i += nch
    alap_ref = refs[i]; i += 1
    W1_ref = refs[i]; i += 1
    b1_ref = refs[i]; i += 1
    W2_ref = refs[i]; i += 1
    b2_ref = refs[i]; i += 1
    ep_ref = refs[i]; i += 1
    o_refs = refs[i:i + 4]

    ep = ep_ref[0, 0]
    acc = jnp.zeros((RBLK, H), dtype=jnp.float32)
    for cidx in range(nch):
        a = a_refs[cidx]
        z = ep * h_refs[cidx][...] + a[0] + a[1]
        w = W1_ref[cidx * 128:(cidx + 1) * 128, :]
        acc = acc + jnp.dot(z, w, preferred_element_type=jnp.float32)
    zlap = ep * lap_ref[...] + alap_ref[0] + alap_ref[1]
    wlap = W1_ref[nch * 128:nch * 128 + K, :]
    acc = acc + jnp.dot(zlap, wlap, preferred_element_type=jnp.float32)
    t = jnp.maximum(acc + b1_ref[...], 0.0)
    o = jnp.dot(t, W2_ref[...], preferred_element_type=jnp.float32) + b2_ref[...]
    o = jnp.maximum(o, 0.0)
    for cidx in range(4):
        oc = o[:, cidx * 128:(cidx + 1) * 128]
        if residual:
            oc = oc + h_refs[cidx][...]
        o_refs[cidx][...] = oc


@functools.partial(jax.jit, static_argnames=("nch", "residual"))
def _mlp(h_chunks, lap, agg_chunks, agglap, W1, b1, W2, b2, ep,
         nch, residual):
    row_spec = pl.BlockSpec((RBLK, 128), lambda i: (i, 0))
    lap_spec = pl.BlockSpec((RBLK, K), lambda i: (i, 0))
    agg_spec = pl.BlockSpec((2, RBLK, 128), lambda i: (0, i, 0))
    alap_spec = pl.BlockSpec((2, RBLK, K), lambda i: (0, i, 0))
    full = lambda shape: pl.BlockSpec(shape, lambda i: tuple(0 for _ in shape))
    smem = pl.BlockSpec(memory_space=pltpu.SMEM)

    in_specs = ([row_spec] * nch + [lap_spec] + [agg_spec] * nch +
                [alap_spec, full(W1.shape), full((1, H)), full(W2.shape),
                 full((1, H)), smem])
    out_specs = [row_spec] * 4
    out_shape = [jax.ShapeDtypeStruct((N, 128), jnp.float32)] * 4

    return pl.pallas_call(
        functools.partial(_mlp_body, nch, residual),
        grid=(NBLK,),
        in_specs=in_specs,
        out_specs=out_specs,
        out_shape=out_shape,
    )(*h_chunks, lap, *agg_chunks, agglap, W1, b1.reshape(1, H),
      W2, b2.reshape(1, H), ep)


def _pool_body(h0, h1, h2, h3, b_ref, Wp_ref, bp_ref, out_ref, psum, cnt):
    i = pl.program_id(0)

    @pl.when(i == 0)
    def _init():
        psum[...] = jnp.zeros_like(psum)
        cnt[...] = jnp.zeros_like(cnt)

    batch = b_ref[0, 0, :]
    ids = lax.broadcasted_iota(jnp.int32, (G, RBLK), 0)
    mask = (batch[None, :] == ids).astype(jnp.float32)
    hcat = jnp.concatenate([h0[...], h1[...], h2[...], h3[...]], axis=1)
    psum[...] += jnp.dot(mask, hcat, preferred_element_type=jnp.float32)
    cnt[...] += jnp.sum(mask, axis=1, keepdims=True)

    @pl.when(i == NBLK - 1)
    def _final():
        pooled = psum[...] / jnp.maximum(cnt[...], 1.0)
        out_ref[...] = (jnp.dot(pooled, Wp_ref[...],
                                preferred_element_type=jnp.float32)
                        + bp_ref[...])


@jax.jit
def _pool(h_chunks, batch, Wp, bp):
    row_spec = pl.BlockSpec((RBLK, 128), lambda i: (i, 0))
    batchr = batch.reshape(NBLK, 1, RBLK)
    full = lambda shape: pl.BlockSpec(shape, lambda i: tuple(0 for _ in shape))
    return pl.pallas_call(
        _pool_body,
        grid=(NBLK,),
        in_specs=[row_spec] * 4 + [
            pl.BlockSpec((1, 1, RBLK), lambda i: (i, 0, 0)),
            full(Wp.shape), full((1, C))],
        out_specs=full((G, C)),
        out_shape=jax.ShapeDtypeStruct((G, C), jnp.float32),
        scratch_shapes=[pltpu.VMEM((G, H), jnp.float32),
                        pltpu.VMEM((G, 1), jnp.float32)],
    )(*h_chunks, batchr, Wp, bp.reshape(1, C))


def kernel(x, edge_index, laplace_feats, batch,
           W1_0, b1_0, W2_0, b2_0, eps_0,
           W1_1, b1_1, W2_1, b2_1, eps_1,
           W1_2, b1_2, W2_2, b2_2, eps_2,
           Wp, bp):
    pad = EPW_P - EPW
    sink_dst = jnp.broadcast_to(SINK + (jnp.arange(pad, dtype=jnp.int32) % 256),
                                (NW, pad))
    src = jnp.pad(edge_index[0].reshape(NW, EPW), ((0, 0), (0, pad)),
                  constant_values=0).reshape(NW, NGROUP, GROUP, STEP)
    dst = jnp.concatenate([edge_index[1].reshape(NW, EPW), sink_dst],
                          axis=1).reshape(NW, NGROUP, GROUP, STEP)
    z128 = jnp.zeros((ROWS_A, 128), dtype=jnp.float32)

    lappad = jnp.pad(laplace_feats, ((0, 0), (0, 128 - K)))
    agglap = _sc_spmm(lappad, src, dst, z128, fc=128)[:, :, :K]
    aggx = _sc_spmm(x, src, dst, z128, fc=128)

    ep0 = jnp.reshape(1.0 + eps_0, (1, 1))
    h1 = _mlp([x], laplace_feats, [aggx], agglap,
              W1_0, b1_0, W2_0, b2_0, ep0, nch=1, residual=False)

    agg1 = [_sc_spmm(h1[c], src, dst, z128, fc=128) for c in range(4)]
    ep1 = jnp.reshape(1.0 + eps_1, (1, 1))
    h2 = _mlp(h1, laplace_feats, agg1, agglap,
              W1_1, b1_1, W2_1, b2_1, ep1, nch=4, residual=True)

    agg2 = [_sc_spmm(h2[c], src, dst, z128, fc=128) for c in range(4)]
    ep2 = jnp.reshape(1.0 + eps_2, (1, 1))
    h3 = _mlp(h2, laplace_feats, agg2, agglap,
              W1_2, b1_2, W2_2, b2_2, ep2, nch=4, residual=True)

    return _pool(h3, batch, Wp, bp)


# speedup vs baseline: 3.0737x; 3.0737x over previous
import functools

import jax
import jax.numpy as jnp
from jax import lax
from jax.experimental import pallas as pl
from jax.experimental.pallas import tpu as pltpu
from jax.experimental.pallas import tpu_sc as plsc

N = 10000
E = 320000
D = 128
K = 16
H = 512
C = 10
G = 64

NUM_CORES = 2
NUM_SUBCORES = 16
NW = NUM_CORES * NUM_SUBCORES
EPW = E // NW
STEP = 100
NSTEP = EPW // STEP
GROUP = 20
NGROUP = NSTEP // GROUP
ROWS_A = 624
TAIL = N - NUM_SUBCORES * ROWS_A
TAIL0 = NUM_SUBCORES * ROWS_A


@jax.jit
def _sc_spmm_multi(tables, srcr, dstr, zeros):
    ntab = len(tables)
    mesh = plsc.VectorSubcoreMesh(core_axis_name="c", subcore_axis_name="s")
    fc = 128

    @functools.partial(
        pl.kernel,
        mesh=mesh,
        out_type=[jax.ShapeDtypeStruct((NUM_CORES, N, fc), jnp.float32)] * ntab,
        scratch_types=[
            pltpu.VMEM((GROUP, STEP), jnp.int32),
            pltpu.VMEM((GROUP, STEP), jnp.int32),
            pltpu.VMEM((2, STEP, fc), jnp.float32),
            pltpu.VMEM_SHARED((N, fc), jnp.float32),
            pltpu.SemaphoreType.DMA,
            pltpu.SemaphoreType.DMA,
        ],
    )
    def k(*refs):
        table_refs = refs[:ntab]
        src_hbm, dst_hbm, zeros_hbm = refs[ntab:ntab + 3]
        out_refs = refs[ntab + 3:2 * ntab + 3]
        sidx, didx, rows, acc_ref, sem0, sem1 = refs[2 * ntab + 3:]
        c = lax.axis_index("c")
        s = lax.axis_index("s")
        wid = c * NUM_SUBCORES + s
        r0 = s * ROWS_A

        for t in range(ntab):
            table_hbm = table_refs[t]
            pltpu.sync_copy(zeros_hbm.at[pl.ds(0, ROWS_A)],
                            acc_ref.at[pl.ds(r0, ROWS_A)])

            @pl.when(s == NUM_SUBCORES - 1)
            def _ztail():
                pltpu.sync_copy(zeros_hbm.at[pl.ds(0, TAIL)],
                                acc_ref.at[pl.ds(TAIL0, TAIL)])

            plsc.subcore_barrier()

            def group(g, carry):
                pltpu.sync_copy(src_hbm.at[wid, g], sidx)
                pltpu.sync_copy(dst_hbm.at[wid, g], didx)
                pltpu.async_copy(table_hbm.at[sidx.at[0]], rows.at[0], sem0)

                def pair(p, carry2):
                    j = 2 * p
                    pltpu.async_copy(table_hbm.at[sidx.at[j + 1]], rows.at[1],
                                     sem1)
                    pltpu.make_async_copy(table_hbm.at[sidx.at[j]],
                                          rows.at[0], sem0).wait()
                    pltpu.sync_copy(rows.at[0], acc_ref.at[didx.at[j]],
                                    add=True)

                    @pl.when(j + 2 < GROUP)
                    def _next():
                        pltpu.async_copy(table_hbm.at[sidx.at[j + 2]],
                                         rows.at[0], sem0)

                    pltpu.make_async_copy(table_hbm.at[sidx.at[j + 1]],
                                          rows.at[1], sem1).wait()
                    pltpu.sync_copy(rows.at[1], acc_ref.at[didx.at[j + 1]],
                                    add=True)
                    return carry2

                lax.fori_loop(0, GROUP // 2, pair, 0)
                return carry

            lax.fori_loop(0, NGROUP, group, 0)
            plsc.subcore_barrier()
            pltpu.sync_copy(acc_ref.at[pl.ds(r0, ROWS_A)],
                            out_refs[t].at[c, pl.ds(r0, ROWS_A)])

            @pl.when(s == NUM_SUBCORES - 1)
            def _otail():
                pltpu.sync_copy(acc_ref.at[pl.ds(TAIL0, TAIL)],
                                out_refs[t].at[c, pl.ds(TAIL0, TAIL)])

    return k(*tables, srcr, dstr, zeros)


RBLK = 400
NBLK = N // RBLK


def _mlp_body(nch, residual, *refs):
    i = 0
    h_refs = refs[i:i + nch]; i += nch
    lap_ref = refs[i]; i += 1
    a_refs = refs[i:i + nch]; i += nch
    alap_ref = refs[i]; i += 1
    W1_ref = refs[i]; i += 1
    b1_ref = refs[i]; i += 1
    W2_ref = refs[i]; i += 1
    b2_ref = refs[i]; i += 1
    ep_ref = refs[i]; i += 1
    o_refs = refs[i:i + 4]

    ep = ep_ref[0, 0]
    acc = jnp.zeros((RBLK, H), dtype=jnp.float32)
    for cidx in range(nch):
        a = a_refs[cidx]
        z = ep * h_refs[cidx][...] + a[0] + a[1]
        w = W1_ref[cidx * 128:(cidx + 1) * 128, :]
        acc = acc + jnp.dot(z, w, preferred_element_type=jnp.float32)
    zlap = ep * lap_ref[...] + alap_ref[0] + alap_ref[1]
    wlap = W1_ref[nch * 128:nch * 128 + K, :]
    acc = acc + jnp.dot(zlap, wlap, preferred_element_type=jnp.float32)
    t = jnp.maximum(acc + b1_ref[...], 0.0)
    o = jnp.dot(t, W2_ref[...], preferred_element_type=jnp.float32) + b2_ref[...]
    o = jnp.maximum(o, 0.0)
    for cidx in range(4):
        oc = o[:, cidx * 128:(cidx + 1) * 128]
        if residual:
            oc = oc + h_refs[cidx][...]
        o_refs[cidx][...] = oc


@functools.partial(jax.jit, static_argnames=("nch", "residual"))
def _mlp(h_chunks, lap, agg_chunks, agglap, W1, b1, W2, b2, ep,
         nch, residual):
    row_spec = pl.BlockSpec((RBLK, 128), lambda i: (i, 0))
    lap_spec = pl.BlockSpec((RBLK, K), lambda i: (i, 0))
    agg_spec = pl.BlockSpec((2, RBLK, 128), lambda i: (0, i, 0))
    alap_spec = pl.BlockSpec((2, RBLK, K), lambda i: (0, i, 0))
    full = lambda shape: pl.BlockSpec(shape, lambda i: tuple(0 for _ in shape))
    smem = pl.BlockSpec(memory_space=pltpu.SMEM)

    in_specs = ([row_spec] * nch + [lap_spec] + [agg_spec] * nch +
                [alap_spec, full(W1.shape), full((1, H)), full(W2.shape),
                 full((1, H)), smem])
    out_specs = [row_spec] * 4
    out_shape = [jax.ShapeDtypeStruct((N, 128), jnp.float32)] * 4

    return pl.pallas_call(
        functools.partial(_mlp_body, nch, residual),
        grid=(NBLK,),
        in_specs=in_specs,
        out_specs=out_specs,
        out_shape=out_shape,
    )(*h_chunks, lap, *agg_chunks, agglap, W1, b1.reshape(1, H),
      W2, b2.reshape(1, H), ep)


def _pool_body(h0, h1, h2, h3, b_ref, Wp_ref, bp_ref, out_ref, psum, cnt):
    i = pl.program_id(0)

    @pl.when(i == 0)
    def _init():
        psum[...] = jnp.zeros_like(psum)
        cnt[...] = jnp.zeros_like(cnt)

    batch = b_ref[0, 0, :]
    ids = lax.broadcasted_iota(jnp.int32, (G, RBLK), 0)
    mask = (batch[None, :] == ids).astype(jnp.float32)
    hcat = jnp.concatenate([h0[...], h1[...], h2[...], h3[...]], axis=1)
    psum[...] += jnp.dot(mask, hcat, preferred_element_type=jnp.float32)
    cnt[...] += jnp.sum(mask, axis=1, keepdims=True)

    @pl.when(i == NBLK - 1)
    def _final():
        pooled = psum[...] / jnp.maximum(cnt[...], 1.0)
        out_ref[...] = (jnp.dot(pooled, Wp_ref[...],
                                preferred_element_type=jnp.float32)
                        + bp_ref[...])


@jax.jit
def _pool(h_chunks, batch, Wp, bp):
    row_spec = pl.BlockSpec((RBLK, 128), lambda i: (i, 0))
    batchr = batch.reshape(NBLK, 1, RBLK)
    full = lambda shape: pl.BlockSpec(shape, lambda i: tuple(0 for _ in shape))
    return pl.pallas_call(
        _pool_body,
        grid=(NBLK,),
        in_specs=[row_spec] * 4 + [
            pl.BlockSpec((1, 1, RBLK), lambda i: (i, 0, 0)),
            full(Wp.shape), full((1, C))],
        out_specs=full((G, C)),
        out_shape=jax.ShapeDtypeStruct((G, C), jnp.float32),
        scratch_shapes=[pltpu.VMEM((G, H), jnp.float32),
                        pltpu.VMEM((G, 1), jnp.float32)],
    )(*h_chunks, batchr, Wp, bp.reshape(1, C))


def kernel(x, edge_index, laplace_feats, batch,
           W1_0, b1_0, W2_0, b2_0, eps_0,
           W1_1, b1_1, W2_1, b2_1, eps_1,
           W1_2, b1_2, W2_2, b2_2, eps_2,
           Wp, bp):
    src = edge_index[0].reshape(NW, NGROUP, GROUP, STEP)
    dst = edge_index[1].reshape(NW, NGROUP, GROUP, STEP)
    z128 = jnp.zeros((ROWS_A, 128), dtype=jnp.float32)

    lappad = jnp.pad(laplace_feats, ((0, 0), (0, 128 - K)))
    aggx, agglap_p = _sc_spmm_multi([x, lappad], src, dst, z128)
    agglap = agglap_p[:, :, :K]

    ep0 = jnp.reshape(1.0 + eps_0, (1, 1))
    h1 = _mlp([x], laplace_feats, [aggx], agglap,
              W1_0, b1_0, W2_0, b2_0, ep0, nch=1, residual=False)

    agg1 = _sc_spmm_multi(h1, src, dst, z128)
    ep1 = jnp.reshape(1.0 + eps_1, (1, 1))
    h2 = _mlp(h1, laplace_feats, agg1, agglap,
              W1_1, b1_1, W2_1, b2_1, ep1, nch=4, residual=True)

    agg2 = _sc_spmm_multi(h2, src, dst, z128)
    ep2 = jnp.reshape(1.0 + eps_2, (1, 1))
    h3 = _mlp(h2, laplace_feats, agg2, agglap,
              W1_2, b1_2, W2_2, b2_2, ep2, nch=4, residual=True)

    return _pool(h3, batch, Wp, bp)


# speedup vs baseline: 3.1983x; 1.0405x over previous
import functools

import jax
import jax.numpy as jnp
from jax import lax
from jax.experimental import pallas as pl
from jax.experimental.pallas import tpu as pltpu
from jax.experimental.pallas import tpu_sc as plsc

N = 10000
E = 320000
D = 128
K = 16
H = 512
C = 10
G = 64

NUM_CORES = 2
NUM_SUBCORES = 16
NW = NUM_CORES * NUM_SUBCORES
EPW = E // NW
STEP = 100
NSTEP = EPW // STEP
GROUP = 20
NGROUP = NSTEP // GROUP
ROWS_A = 624
TAIL = N - NUM_SUBCORES * ROWS_A
TAIL0 = NUM_SUBCORES * ROWS_A


@jax.jit
def _sc_spmm_multi(tables, sd_idx, zeros):
    ntab = len(tables)
    mesh = plsc.VectorSubcoreMesh(core_axis_name="c", subcore_axis_name="s")
    fc = 128

    @functools.partial(
        pl.kernel,
        mesh=mesh,
        out_type=[jax.ShapeDtypeStruct((NUM_CORES, N, fc), jnp.float32)] * ntab,
        scratch_types=[
            pltpu.VMEM((2, 2, GROUP, STEP), jnp.int32),
            pltpu.VMEM((2, STEP, fc), jnp.float32),
            pltpu.VMEM_SHARED((N, fc), jnp.float32),
            pltpu.SemaphoreType.DMA,
            pltpu.SemaphoreType.DMA,
            pltpu.SemaphoreType.DMA,
            pltpu.SemaphoreType.DMA,
        ],
    )
    def k(*refs):
        table_refs = refs[:ntab]
        sd_hbm, zeros_hbm = refs[ntab:ntab + 2]
        out_refs = refs[ntab + 2:2 * ntab + 2]
        sd, rows, acc_ref, sem0, sem1, isem0, isem1 = refs[2 * ntab + 2:]
        isems = (isem0, isem1)
        c = lax.axis_index("c")
        s = lax.axis_index("s")
        wid = c * NUM_SUBCORES + s
        r0 = s * ROWS_A

        for t in range(ntab):
            table_hbm = table_refs[t]
            pltpu.async_copy(sd_hbm.at[wid, 0], sd.at[0], isem0)
            pltpu.sync_copy(zeros_hbm.at[pl.ds(0, ROWS_A)],
                            acc_ref.at[pl.ds(r0, ROWS_A)])

            @pl.when(s == NUM_SUBCORES - 1)
            def _ztail():
                pltpu.sync_copy(zeros_hbm.at[pl.ds(0, TAIL)],
                                acc_ref.at[pl.ds(TAIL0, TAIL)])

            plsc.subcore_barrier()

            for g in range(NGROUP):
                gb = g % 2
                pltpu.make_async_copy(sd_hbm.at[wid, g], sd.at[gb],
                                      isems[gb]).wait()
                if g + 1 < NGROUP:
                    pltpu.async_copy(sd_hbm.at[wid, g + 1], sd.at[1 - gb],
                                     isems[1 - gb])
                sidx = sd.at[gb, 0]
                didx = sd.at[gb, 1]
                pltpu.async_copy(table_hbm.at[sidx.at[0]], rows.at[0], sem0)

                def pair(p, carry2):
                    j = 2 * p
                    pltpu.async_copy(table_hbm.at[sidx.at[j + 1]], rows.at[1],
                                     sem1)
                    pltpu.make_async_copy(table_hbm.at[sidx.at[j]],
                                          rows.at[0], sem0).wait()
                    pltpu.sync_copy(rows.at[0], acc_ref.at[didx.at[j]],
                                    add=True)

                    @pl.when(j + 2 < GROUP)
                    def _next():
                        pltpu.async_copy(table_hbm.at[sidx.at[j + 2]],
                                         rows.at[0], sem0)

                    pltpu.make_async_copy(table_hbm.at[sidx.at[j + 1]],
                                          rows.at[1], sem1).wait()
                    pltpu.sync_copy(rows.at[1], acc_ref.at[didx.at[j + 1]],
                                    add=True)
                    return carry2

                lax.fori_loop(0, GROUP // 2, pair, 0)

            plsc.subcore_barrier()
            pltpu.sync_copy(acc_ref.at[pl.ds(r0, ROWS_A)],
                            out_refs[t].at[c, pl.ds(r0, ROWS_A)])

            @pl.when(s == NUM_SUBCORES - 1)
            def _otail():
                pltpu.sync_copy(acc_ref.at[pl.ds(TAIL0, TAIL)],
                                out_refs[t].at[c, pl.ds(TAIL0, TAIL)])

    return k(*tables, sd_idx, zeros)


RBLK = 400
NBLK = N // RBLK


def _mlp_body(nch, residual, *refs):
    i = 0
    h_refs = refs[i:i + nch]; i += nch
    lap_ref = refs[i]; i += 1
    a_refs = refs[i:i + nch]; i += nch
    alap_ref = refs[i]; i += 1
    W1_ref = refs[i]; i += 1
    b1_ref = refs[i]; i += 1
    W2_ref = refs[i]; i += 1
    b2_ref = refs[i]; i += 1
    ep_ref = refs[i]; i += 1
    o_refs = refs[i:i + 4]

    ep = ep_ref[0, 0]
    acc = jnp.zeros((RBLK, H), dtype=jnp.float32)
    for cidx in range(nch):
        a = a_refs[cidx]
        z = ep * h_refs[cidx][...] + a[0] + a[1]
        w = W1_ref[cidx * 128:(cidx + 1) * 128, :]
        acc = acc + jnp.dot(z, w, preferred_element_type=jnp.float32)
    zlap = ep * lap_ref[...] + alap_ref[0] + alap_ref[1]
    wlap = W1_ref[nch * 128:nch * 128 + K, :]
    acc = acc + jnp.dot(zlap, wlap, preferred_element_type=jnp.float32)
    t = jnp.maximum(acc + b1_ref[...], 0.0)
    o = jnp.dot(t, W2_ref[...], preferred_element_type=jnp.float32) + b2_ref[...]
    o = jnp.maximum(o, 0.0)
    for cidx in range(4):
        oc = o[:, cidx * 128:(cidx + 1) * 128]
        if residual:
            oc = oc + h_refs[cidx][...]
        o_refs[cidx][...] = oc


@functools.partial(jax.jit, static_argnames=("nch", "residual"))
def _mlp(h_chunks, lap, agg_chunks, agglap, W1, b1, W2, b2, ep,
         nch, residual):
    row_spec = pl.BlockSpec((RBLK, 128), lambda i: (i, 0))
    lap_spec = pl.BlockSpec((RBLK, K), lambda i: (i, 0))
    agg_spec = pl.BlockSpec((2, RBLK, 128), lambda i: (0, i, 0))
    alap_spec = pl.BlockSpec((2, RBLK, K), lambda i: (0, i, 0))
    full = lambda shape: pl.BlockSpec(shape, lambda i: tuple(0 for _ in shape))
    smem = pl.BlockSpec(memory_space=pltpu.SMEM)

    in_specs = ([row_spec] * nch + [lap_spec] + [agg_spec] * nch +
                [alap_spec, full(W1.shape), full((1, H)), full(W2.shape),
                 full((1, H)), smem])
    out_specs = [row_spec] * 4
    out_shape = [jax.ShapeDtypeStruct((N, 128), jnp.float32)] * 4

    return pl.pallas_call(
        functools.partial(_mlp_body, nch, residual),
        grid=(NBLK,),
        in_specs=in_specs,
        out_specs=out_specs,
        out_shape=out_shape,
    )(*h_chunks, lap, *agg_chunks, agglap, W1, b1.reshape(1, H),
      W2, b2.reshape(1, H), ep)


def _pool_body(h0, h1, h2, h3, b_ref, Wp_ref, bp_ref, out_ref, psum, cnt):
    i = pl.program_id(0)

    @pl.when(i == 0)
    def _init():
        psum[...] = jnp.zeros_like(psum)
        cnt[...] = jnp.zeros_like(cnt)

    batch = b_ref[0, 0, :]
    ids = lax.broadcasted_iota(jnp.int32, (G, RBLK), 0)
    mask = (batch[None, :] == ids).astype(jnp.float32)
    hcat = jnp.concatenate([h0[...], h1[...], h2[...], h3[...]], axis=1)
    psum[...] += jnp.dot(mask, hcat, preferred_element_type=jnp.float32)
    cnt[...] += jnp.sum(mask, axis=1, keepdims=True)

    @pl.when(i == NBLK - 1)
    def _final():
        pooled = psum[...] / jnp.maximum(cnt[...], 1.0)
        out_ref[...] = (jnp.dot(pooled, Wp_ref[...],
                                preferred_element_type=jnp.float32)
                        + bp_ref[...])


@jax.jit
def _pool(h_chunks, batch, Wp, bp):
    row_spec = pl.BlockSpec((RBLK, 128), lambda i: (i, 0))
    batchr = batch.reshape(NBLK, 1, RBLK)
    full = lambda shape: pl.BlockSpec(shape, lambda i: tuple(0 for _ in shape))
    return pl.pallas_call(
        _pool_body,
        grid=(NBLK,),
        in_specs=[row_spec] * 4 + [
            pl.BlockSpec((1, 1, RBLK), lambda i: (i, 0, 0)),
            full(Wp.shape), full((1, C))],
        out_specs=full((G, C)),
        out_shape=jax.ShapeDtypeStruct((G, C), jnp.float32),
        scratch_shapes=[pltpu.VMEM((G, H), jnp.float32),
                        pltpu.VMEM((G, 1), jnp.float32)],
    )(*h_chunks, batchr, Wp, bp.reshape(1, C))


def kernel(x, edge_index, laplace_feats, batch,
           W1_0, b1_0, W2_0, b2_0, eps_0,
           W1_1, b1_1, W2_1, b2_1, eps_1,
           W1_2, b1_2, W2_2, b2_2, eps_2,
           Wp, bp):
    srcr = edge_index[0].reshape(NW, NGROUP, 1, GROUP, STEP)
    dstr = edge_index[1].reshape(NW, NGROUP, 1, GROUP, STEP)
    sd = jnp.concatenate([srcr, dstr], axis=2)
    z128 = jnp.zeros((ROWS_A, 128), dtype=jnp.float32)

    lappad = jnp.pad(laplace_feats, ((0, 0), (0, 128 - K)))
    aggx, agglap_p = _sc_spmm_multi([x, lappad], sd, z128)
    agglap = agglap_p[:, :, :K]

    ep0 = jnp.reshape(1.0 + eps_0, (1, 1))
    h1 = _mlp([x], laplace_feats, [aggx], agglap,
              W1_0, b1_0, W2_0, b2_0, ep0, nch=1, residual=False)

    agg1 = _sc_spmm_multi(h1, sd, z128)
    ep1 = jnp.reshape(1.0 + eps_1, (1, 1))
    h2 = _mlp(h1, laplace_feats, agg1, agglap,
              W1_1, b1_1, W2_1, b2_1, ep1, nch=4, residual=True)

    agg2 = _sc_spmm_multi(h2, sd, z128)
    ep2 = jnp.reshape(1.0 + eps_2, (1, 1))
    h3 = _mlp(h2, laplace_feats, agg2, agglap,
              W1_2, b1_2, W2_2, b2_2, ep2, nch=4, residual=True)

    return _pool(h3, batch, Wp, bp)


# speedup vs baseline: 3.2582x; 1.0187x over previous
import functools

import jax
import jax.numpy as jnp
from jax import lax
from jax.experimental import pallas as pl
from jax.experimental.pallas import tpu as pltpu
from jax.experimental.pallas import tpu_sc as plsc

N = 10000
E = 320000
D = 128
K = 16
H = 512
C = 10
G = 64

NUM_CORES = 2
NUM_SUBCORES = 16
NW = NUM_CORES * NUM_SUBCORES
EPW = E // NW
STEP = 100
NSTEP = EPW // STEP
GROUP = 20
NGROUP = NSTEP // GROUP
ROWS_A = 624
TAIL = N - NUM_SUBCORES * ROWS_A
TAIL0 = NUM_SUBCORES * ROWS_A


@jax.jit
def _sc_spmm_multi(tables, sd_idx, zeros):
    ntab = len(tables)
    mesh = plsc.VectorSubcoreMesh(core_axis_name="c", subcore_axis_name="s")
    fc = 128

    @functools.partial(
        pl.kernel,
        mesh=mesh,
        out_type=[jax.ShapeDtypeStruct((NUM_CORES, N, fc), jnp.float32)] * ntab,
        scratch_types=[
            pltpu.VMEM((2, 2, GROUP, STEP), jnp.int32),
            pltpu.VMEM((2, STEP, fc), jnp.float32),
            pltpu.VMEM_SHARED((N, fc), jnp.float32),
            pltpu.SemaphoreType.DMA,
            pltpu.SemaphoreType.DMA,
            pltpu.SemaphoreType.DMA,
            pltpu.SemaphoreType.DMA,
        ],
    )
    def k(*refs):
        table_refs = refs[:ntab]
        sd_hbm, zeros_hbm = refs[ntab:ntab + 2]
        out_refs = refs[ntab + 2:2 * ntab + 2]
        sd, rows, acc_ref, sem0, sem1, isem0, isem1 = refs[2 * ntab + 2:]
        isems = (isem0, isem1)
        c = lax.axis_index("c")
        s = lax.axis_index("s")
        wid = c * NUM_SUBCORES + s
        r0 = s * ROWS_A

        for t in range(ntab):
            table_hbm = table_refs[t]
            pltpu.async_copy(sd_hbm.at[wid, 0], sd.at[0], isem0)
            pltpu.sync_copy(zeros_hbm.at[pl.ds(0, ROWS_A)],
                            acc_ref.at[pl.ds(r0, ROWS_A)])

            @pl.when(s == NUM_SUBCORES - 1)
            def _ztail():
                pltpu.sync_copy(zeros_hbm.at[pl.ds(0, TAIL)],
                                acc_ref.at[pl.ds(TAIL0, TAIL)])

            plsc.subcore_barrier()

            for g in range(NGROUP):
                gb = g % 2
                pltpu.make_async_copy(sd_hbm.at[wid, g], sd.at[gb],
                                      isems[gb]).wait()
                if g + 1 < NGROUP:
                    pltpu.async_copy(sd_hbm.at[wid, g + 1], sd.at[1 - gb],
                                     isems[1 - gb])
                sidx = sd.at[gb, 0]
                didx = sd.at[gb, 1]
                pltpu.async_copy(table_hbm.at[sidx.at[0]], rows.at[0], sem0)

                def pair(p, carry2):
                    j = 2 * p
                    pltpu.async_copy(table_hbm.at[sidx.at[j + 1]], rows.at[1],
                                     sem1)
                    pltpu.make_async_copy(table_hbm.at[sidx.at[j]],
                                          rows.at[0], sem0).wait()
                    pltpu.sync_copy(rows.at[0], acc_ref.at[didx.at[j]],
                                    add=True)

                    @pl.when(j + 2 < GROUP)
                    def _next():
                        pltpu.async_copy(table_hbm.at[sidx.at[j + 2]],
                                         rows.at[0], sem0)

                    pltpu.make_async_copy(table_hbm.at[sidx.at[j + 1]],
                                          rows.at[1], sem1).wait()
                    pltpu.sync_copy(rows.at[1], acc_ref.at[didx.at[j + 1]],
                                    add=True)
                    return carry2

                lax.fori_loop(0, GROUP // 2, pair, 0)

            plsc.subcore_barrier()
            pltpu.sync_copy(acc_ref.at[pl.ds(r0, ROWS_A)],
                            out_refs[t].at[c, pl.ds(r0, ROWS_A)])

            @pl.when(s == NUM_SUBCORES - 1)
            def _otail():
                pltpu.sync_copy(acc_ref.at[pl.ds(TAIL0, TAIL)],
                                out_refs[t].at[c, pl.ds(TAIL0, TAIL)])

    return k(*tables, sd_idx, zeros)


RBLK = 400
NBLK = N // RBLK


def _mlp_body(nch, residual, *refs):
    i = 0
    h_refs = refs[i:i + nch]; i += nch
    lap_ref = refs[i]; i += 1
    a_refs = refs[i:i + nch]; i += nch
    alap_ref = refs[i]; i += 1
    W1_ref = refs[i]; i += 1
    b1_ref = refs[i]; i += 1
    W2_ref = refs[i]; i += 1
    b2_ref = refs[i]; i += 1
    ep_ref = refs[i]; i += 1
    o_refs = refs[i:i + 4]

    ep = ep_ref[0, 0]
    acc = jnp.zeros((RBLK, H), dtype=jnp.float32)
    for cidx in range(nch):
        a = a_refs[cidx]
        z = ep * h_refs[cidx][...] + a[0] + a[1]
        w = W1_ref[cidx * 128:(cidx + 1) * 128, :]
        acc = acc + jnp.dot(z, w, preferred_element_type=jnp.float32)
    zlap = ep * lap_ref[...] + alap_ref[0] + alap_ref[1]
    wlap = W1_ref[nch * 128:nch * 128 + K, :]
    acc = acc + jnp.dot(zlap, wlap, preferred_element_type=jnp.float32)
    t = jnp.maximum(acc + b1_ref[...], 0.0)
    o = jnp.dot(t, W2_ref[...], preferred_element_type=jnp.float32) + b2_ref[...]
    o = jnp.maximum(o, 0.0)
    for cidx in range(4):
        oc = o[:, cidx * 128:(cidx + 1) * 128]
        if residual:
            oc = oc + h_refs[cidx][...]
        o_refs[cidx][...] = oc


@functools.partial(jax.jit, static_argnames=("nch", "residual"))
def _mlp(h_chunks, lap, agg_chunks, agglap, W1, b1, W2, b2, ep,
         nch, residual):
    row_spec = pl.BlockSpec((RBLK, 128), lambda i: (i, 0))
    lap_spec = pl.BlockSpec((RBLK, K), lambda i: (i, 0))
    agg_spec = pl.BlockSpec((2, RBLK, 128), lambda i: (0, i, 0))
    alap_spec = pl.BlockSpec((2, RBLK, K), lambda i: (0, i, 0))
    full = lambda shape: pl.BlockSpec(shape, lambda i: tuple(0 for _ in shape))
    smem = pl.BlockSpec(memory_space=pltpu.SMEM)

    in_specs = ([row_spec] * nch + [lap_spec] + [agg_spec] * nch +
                [alap_spec, full(W1.shape), full((1, H)), full(W2.shape),
                 full((1, H)), smem])
    out_specs = [row_spec] * 4
    out_shape = [jax.ShapeDtypeStruct((N, 128), jnp.float32)] * 4

    return pl.pallas_call(
        functools.partial(_mlp_body, nch, residual),
        grid=(NBLK,),
        in_specs=in_specs,
        out_specs=out_specs,
        out_shape=out_shape,
    )(*h_chunks, lap, *agg_chunks, agglap, W1, b1.reshape(1, H),
      W2, b2.reshape(1, H), ep)


def _mlp_pool_body(*refs):
    h_refs = refs[0:4]
    lap_ref = refs[4]
    a_refs = refs[5:9]
    alap_ref = refs[9]
    W1_ref, b1_ref, W2_ref, b2_ref, ep_ref, b_ref, Wp_ref, bp_ref = refs[10:18]
    out_ref = refs[18]
    psum, cnt = refs[19:21]

    i = pl.program_id(0)

    @pl.when(i == 0)
    def _init():
        psum[...] = jnp.zeros_like(psum)
        cnt[...] = jnp.zeros_like(cnt)

    ep = ep_ref[0, 0]
    acc = jnp.zeros((RBLK, H), dtype=jnp.float32)
    for cidx in range(4):
        a = a_refs[cidx]
        z = (ep * h_refs[cidx][...] + a[0].astype(jnp.float32)
             + a[1].astype(jnp.float32))
        w = W1_ref[cidx * 128:(cidx + 1) * 128, :]
        acc = acc + jnp.dot(z, w, preferred_element_type=jnp.float32)
    zlap = (ep * lap_ref[...] + alap_ref[0].astype(jnp.float32)
            + alap_ref[1].astype(jnp.float32))
    wlap = W1_ref[4 * 128:4 * 128 + K, :]
    acc = acc + jnp.dot(zlap, wlap, preferred_element_type=jnp.float32)
    t = jnp.maximum(acc + b1_ref[...], 0.0)
    o = jnp.dot(t, W2_ref[...], preferred_element_type=jnp.float32) + b2_ref[...]
    o = jnp.maximum(o, 0.0)
    hcat = jnp.concatenate([h_refs[c][...] for c in range(4)], axis=1)
    h3 = o + hcat

    batch = b_ref[0, 0, :]
    ids = lax.broadcasted_iota(jnp.int32, (G, RBLK), 0)
    mask = (batch[None, :] == ids).astype(jnp.float32)
    psum[...] += jnp.dot(mask, h3, preferred_element_type=jnp.float32)
    cnt[...] += jnp.sum(mask, axis=1, keepdims=True)

    @pl.when(i == NBLK - 1)
    def _final():
        pooled = psum[...] / jnp.maximum(cnt[...], 1.0)
        out_ref[...] = (jnp.dot(pooled, Wp_ref[...],
                                preferred_element_type=jnp.float32)
                        + bp_ref[...])


@jax.jit
def _mlp_pool(h_chunks, lap, agg_chunks, agglap, W1, b1, W2, b2, ep,
              batch, Wp, bp):
    row_spec = pl.BlockSpec((RBLK, 128), lambda i: (i, 0))
    lap_spec = pl.BlockSpec((RBLK, K), lambda i: (i, 0))
    agg_spec = pl.BlockSpec((2, RBLK, 128), lambda i: (0, i, 0))
    alap_spec = pl.BlockSpec((2, RBLK, K), lambda i: (0, i, 0))
    full = lambda shape: pl.BlockSpec(shape, lambda i: tuple(0 for _ in shape))
    smem = pl.BlockSpec(memory_space=pltpu.SMEM)
    batchr = batch.reshape(NBLK, 1, RBLK)

    in_specs = ([row_spec] * 4 + [lap_spec] + [agg_spec] * 4 +
                [alap_spec, full(W1.shape), full((1, H)), full(W2.shape),
                 full((1, H)), smem,
                 pl.BlockSpec((1, 1, RBLK), lambda i: (i, 0, 0)),
                 full(Wp.shape), full((1, C))])

    return pl.pallas_call(
        _mlp_pool_body,
        grid=(NBLK,),
        in_specs=in_specs,
        out_specs=full((G, C)),
        out_shape=jax.ShapeDtypeStruct((G, C), jnp.float32),
        scratch_shapes=[pltpu.VMEM((G, H), jnp.float32),
                        pltpu.VMEM((G, 1), jnp.float32)],
    )(*h_chunks, lap, *agg_chunks, agglap, W1, b1.reshape(1, H),
      W2, b2.reshape(1, H), ep, batchr, Wp, bp.reshape(1, C))


def _pool_body(h0, h1, h2, h3, b_ref, Wp_ref, bp_ref, out_ref, psum, cnt):
    i = pl.program_id(0)

    @pl.when(i == 0)
    def _init():
        psum[...] = jnp.zeros_like(psum)
        cnt[...] = jnp.zeros_like(cnt)

    batch = b_ref[0, 0, :]
    ids = lax.broadcasted_iota(jnp.int32, (G, RBLK), 0)
    mask = (batch[None, :] == ids).astype(jnp.float32)
    hcat = jnp.concatenate([h0[...], h1[...], h2[...], h3[...]], axis=1)
    psum[...] += jnp.dot(mask, hcat, preferred_element_type=jnp.float32)
    cnt[...] += jnp.sum(mask, axis=1, keepdims=True)

    @pl.when(i == NBLK - 1)
    def _final():
        pooled = psum[...] / jnp.maximum(cnt[...], 1.0)
        out_ref[...] = (jnp.dot(pooled, Wp_ref[...],
                                preferred_element_type=jnp.float32)
                        + bp_ref[...])


@jax.jit
def _pool(h_chunks, batch, Wp, bp):
    row_spec = pl.BlockSpec((RBLK, 128), lambda i: (i, 0))
    batchr = batch.reshape(NBLK, 1, RBLK)
    full = lambda shape: pl.BlockSpec(shape, lambda i: tuple(0 for _ in shape))
    return pl.pallas_call(
        _pool_body,
        grid=(NBLK,),
        in_specs=[row_spec] * 4 + [
            pl.BlockSpec((1, 1, RBLK), lambda i: (i, 0, 0)),
            full(Wp.shape), full((1, C))],
        out_specs=full((G, C)),
        out_shape=jax.ShapeDtypeStruct((G, C), jnp.float32),
        scratch_shapes=[pltpu.VMEM((G, H), jnp.float32),
                        pltpu.VMEM((G, 1), jnp.float32)],
    )(*h_chunks, batchr, Wp, bp.reshape(1, C))


def kernel(x, edge_index, laplace_feats, batch,
           W1_0, b1_0, W2_0, b2_0, eps_0,
           W1_1, b1_1, W2_1, b2_1, eps_1,
           W1_2, b1_2, W2_2, b2_2, eps_2,
           Wp, bp):
    srcr = edge_index[0].reshape(NW, NGROUP, 1, GROUP, STEP)
    dstr = edge_index[1].reshape(NW, NGROUP, 1, GROUP, STEP)
    sd = jnp.concatenate([srcr, dstr], axis=2)
    z128 = jnp.zeros((ROWS_A, 128), dtype=jnp.float32)

    lappad = jnp.pad(laplace_feats, ((0, 0), (0, 128 - K)))
    aggx, agglap_p = _sc_spmm_multi([x, lappad], sd, z128)
    agglap = agglap_p[:, :, :K]

    ep0 = jnp.reshape(1.0 + eps_0, (1, 1))
    h1 = _mlp([x], laplace_feats, [aggx], agglap,
              W1_0, b1_0, W2_0, b2_0, ep0, nch=1, residual=False)

    agg1 = _sc_spmm_multi(h1, sd, z128)
    ep1 = jnp.reshape(1.0 + eps_1, (1, 1))
    h2 = _mlp(h1, laplace_feats, agg1, agglap,
              W1_1, b1_1, W2_1, b2_1, ep1, nch=4, residual=True)

    agg2 = _sc_spmm_multi(h2, sd, z128)
    ep2 = jnp.reshape(1.0 + eps_2, (1, 1))
    return _mlp_pool(h2, laplace_feats, agg2, agglap,
                     W1_2, b1_2, W2_2, b2_2, ep2, batch, Wp, bp)


# speedup vs baseline: 3.6498x; 1.1202x over previous
import functools

import jax
import jax.numpy as jnp
from jax import lax
from jax.experimental import pallas as pl
from jax.experimental.pallas import tpu as pltpu
from jax.experimental.pallas import tpu_sc as plsc

N = 10000
E = 320000
D = 128
K = 16
H = 512
C = 10
G = 64

NUM_CORES = 2
NUM_SUBCORES = 16
NW = NUM_CORES * NUM_SUBCORES
EPW = E // NW
STEP = 50
NSTEP = EPW // STEP
GROUP = 40
NGROUP = NSTEP // GROUP
ROWS_A = 624
TAIL = N - NUM_SUBCORES * ROWS_A
TAIL0 = NUM_SUBCORES * ROWS_A


@jax.jit
def _sc_spmm_multi(tables, sd_idx, zeros):
    ntab = len(tables)
    mesh = plsc.VectorSubcoreMesh(core_axis_name="c", subcore_axis_name="s")
    fc = 128

    @functools.partial(
        pl.kernel,
        mesh=mesh,
        out_type=[jax.ShapeDtypeStruct((NUM_CORES, N, fc), jnp.float32)] * ntab,
        scratch_types=[
            pltpu.VMEM((2, 2, GROUP, STEP), jnp.int32),
            pltpu.VMEM((4, STEP, fc), jnp.float32),
            pltpu.VMEM_SHARED((N, fc), jnp.float32),
        ] + [pltpu.SemaphoreType.DMA] * 6,
    )
    def k(*refs):
        table_refs = refs[:ntab]
        sd_hbm, zeros_hbm = refs[ntab:ntab + 2]
        out_refs = refs[ntab + 2:2 * ntab + 2]
        (sd, rows, acc_ref, sem0, sem1, sem2, sem3,
         isem0, isem1) = refs[2 * ntab + 2:]
        sems = (sem0, sem1, sem2, sem3)
        isems = (isem0, isem1)
        c = lax.axis_index("c")
        s = lax.axis_index("s")
        wid = c * NUM_SUBCORES + s
        r0 = s * ROWS_A

        for t in range(ntab):
            table_hbm = table_refs[t]
            pltpu.async_copy(sd_hbm.at[wid, 0], sd.at[0], isem0)
            pltpu.sync_copy(zeros_hbm.at[pl.ds(0, ROWS_A)],
                            acc_ref.at[pl.ds(r0, ROWS_A)])

            @pl.when(s == NUM_SUBCORES - 1)
            def _ztail():
                pltpu.sync_copy(zeros_hbm.at[pl.ds(0, TAIL)],
                                acc_ref.at[pl.ds(TAIL0, TAIL)])

            plsc.subcore_barrier()

            for g in range(NGROUP):
                gb = g % 2
                pltpu.make_async_copy(sd_hbm.at[wid, g], sd.at[gb],
                                      isems[gb]).wait()
                if g + 1 < NGROUP:
                    pltpu.async_copy(sd_hbm.at[wid, g + 1], sd.at[1 - gb],
                                     isems[1 - gb])
                sidx = sd.at[gb, 0]
                didx = sd.at[gb, 1]
                for b in range(3):
                    pltpu.async_copy(table_hbm.at[sidx.at[b]], rows.at[b],
                                     sems[b])

                def quad(q, carry2):
                    j0 = 4 * q
                    for b in range(4):
                        j = j0 + b
                        jn = j + 3
                        bn = (b + 3) % 4

                        @pl.when(jn < GROUP)
                        def _next():
                            pltpu.async_copy(table_hbm.at[sidx.at[jn]],
                                             rows.at[bn], sems[bn])

                        pltpu.make_async_copy(table_hbm.at[sidx.at[j]],
                                              rows.at[b], sems[b]).wait()
                        pltpu.sync_copy(rows.at[b], acc_ref.at[didx.at[j]],
                                        add=True)
                    return carry2

                lax.fori_loop(0, GROUP // 4, quad, 0)

            plsc.subcore_barrier()
            pltpu.sync_copy(acc_ref.at[pl.ds(r0, ROWS_A)],
                            out_refs[t].at[c, pl.ds(r0, ROWS_A)])

            @pl.when(s == NUM_SUBCORES - 1)
            def _otail():
                pltpu.sync_copy(acc_ref.at[pl.ds(TAIL0, TAIL)],
                                out_refs[t].at[c, pl.ds(TAIL0, TAIL)])

    return k(*tables, sd_idx, zeros)


RBLK = 400
NBLK = N // RBLK


def _mlp_body(nch, residual, *refs):
    i = 0
    h_refs = refs[i:i + nch]; i += nch
    lap_ref = refs[i]; i += 1
    a_refs = refs[i:i + nch]; i += nch
    alap_ref = refs[i]; i += 1
    W1_ref = refs[i]; i += 1
    b1_ref = refs[i]; i += 1
    W2_ref = refs[i]; i += 1
    b2_ref = refs[i]; i += 1
    ep_ref = refs[i]; i += 1
    o_refs = refs[i:i + 4]

    ep = ep_ref[0, 0]
    acc = jnp.zeros((RBLK, H), dtype=jnp.float32)
    for cidx in range(nch):
        a = a_refs[cidx]
        z = ep * h_refs[cidx][...] + a[0] + a[1]
        w = W1_ref[cidx * 128:(cidx + 1) * 128, :]
        acc = acc + jnp.dot(z, w, preferred_element_type=jnp.float32)
    zlap = ep * lap_ref[...] + alap_ref[0] + alap_ref[1]
    wlap = W1_ref[nch * 128:nch * 128 + K, :]
    acc = acc + jnp.dot(zlap, wlap, preferred_element_type=jnp.float32)
    t = jnp.maximum(acc + b1_ref[...], 0.0)
    o = jnp.dot(t, W2_ref[...], preferred_element_type=jnp.float32) + b2_ref[...]
    o = jnp.maximum(o, 0.0)
    for cidx in range(4):
        oc = o[:, cidx * 128:(cidx + 1) * 128]
        if residual:
            oc = oc + h_refs[cidx][...]
        o_refs[cidx][...] = oc


@functools.partial(jax.jit, static_argnames=("nch", "residual"))
def _mlp(h_chunks, lap, agg_chunks, agglap, W1, b1, W2, b2, ep,
         nch, residual):
    row_spec = pl.BlockSpec((RBLK, 128), lambda i: (i, 0))
    lap_spec = pl.BlockSpec((RBLK, K), lambda i: (i, 0))
    agg_spec = pl.BlockSpec((2, RBLK, 128), lambda i: (0, i, 0))
    alap_spec = pl.BlockSpec((2, RBLK, K), lambda i: (0, i, 0))
    full = lambda shape: pl.BlockSpec(shape, lambda i: tuple(0 for _ in shape))
    smem = pl.BlockSpec(memory_space=pltpu.SMEM)

    in_specs = ([row_spec] * nch + [lap_spec] + [agg_spec] * nch +
                [alap_spec, full(W1.shape), full((1, H)), full(W2.shape),
                 full((1, H)), smem])
    out_specs = [row_spec] * 4
    out_shape = [jax.ShapeDtypeStruct((N, 128), jnp.float32)] * 4

    return pl.pallas_call(
        functools.partial(_mlp_body, nch, residual),
        grid=(NBLK,),
        in_specs=in_specs,
        out_specs=out_specs,
        out_shape=out_shape,
    )(*h_chunks, lap, *agg_chunks, agglap, W1, b1.reshape(1, H),
      W2, b2.reshape(1, H), ep)


def _mlp_pool_body(*refs):
    h_refs = refs[0:4]
    lap_ref = refs[4]
    a_refs = refs[5:9]
    alap_ref = refs[9]
    W1_ref, b1_ref, W2_ref, b2_ref, ep_ref, b_ref, Wp_ref, bp_ref = refs[10:18]
    out_ref = refs[18]
    psum, cnt = refs[19:21]

    i = pl.program_id(0)

    @pl.when(i == 0)
    def _init():
        psum[...] = jnp.zeros_like(psum)
        cnt[...] = jnp.zeros_like(cnt)

    ep = ep_ref[0, 0]
    acc = jnp.zeros((RBLK, H), dtype=jnp.float32)
    for cidx in range(4):
        a = a_refs[cidx]
        z = (ep * h_refs[cidx][...] + a[0].astype(jnp.float32)
             + a[1].astype(jnp.float32))
        w = W1_ref[cidx * 128:(cidx + 1) * 128, :]
        acc = acc + jnp.dot(z, w, preferred_element_type=jnp.float32)
    zlap = (ep * lap_ref[...] + alap_ref[0].astype(jnp.float32)
            + alap_ref[1].astype(jnp.float32))
    wlap = W1_ref[4 * 128:4 * 128 + K, :]
    acc = acc + jnp.dot(zlap, wlap, preferred_element_type=jnp.float32)
    t = jnp.maximum(acc + b1_ref[...], 0.0)
    o = jnp.dot(t, W2_ref[...], preferred_element_type=jnp.float32) + b2_ref[...]
    o = jnp.maximum(o, 0.0)
    hcat = jnp.concatenate([h_refs[c][...] for c in range(4)], axis=1)
    h3 = o + hcat

    batch = b_ref[0, 0, :]
    ids = lax.broadcasted_iota(jnp.int32, (G, RBLK), 0)
    mask = (batch[None, :] == ids).astype(jnp.float32)
    psum[...] += jnp.dot(mask, h3, preferred_element_type=jnp.float32)
    cnt[...] += jnp.sum(mask, axis=1, keepdims=True)

    @pl.when(i == NBLK - 1)
    def _final():
        pooled = psum[...] / jnp.maximum(cnt[...], 1.0)
        out_ref[...] = (jnp.dot(pooled, Wp_ref[...],
                                preferred_element_type=jnp.float32)
                        + bp_ref[...])


@jax.jit
def _mlp_pool(h_chunks, lap, agg_chunks, agglap, W1, b1, W2, b2, ep,
              batch, Wp, bp):
    row_spec = pl.BlockSpec((RBLK, 128), lambda i: (i, 0))
    lap_spec = pl.BlockSpec((RBLK, K), lambda i: (i, 0))
    agg_spec = pl.BlockSpec((2, RBLK, 128), lambda i: (0, i, 0))
    alap_spec = pl.BlockSpec((2, RBLK, K), lambda i: (0, i, 0))
    full = lambda shape: pl.BlockSpec(shape, lambda i: tuple(0 for _ in shape))
    smem = pl.BlockSpec(memory_space=pltpu.SMEM)
    batchr = batch.reshape(NBLK, 1, RBLK)

    in_specs = ([row_spec] * 4 + [lap_spec] + [agg_spec] * 4 +
                [alap_spec, full(W1.shape), full((1, H)), full(W2.shape),
                 full((1, H)), smem,
                 pl.BlockSpec((1, 1, RBLK), lambda i: (i, 0, 0)),
                 full(Wp.shape), full((1, C))])

    return pl.pallas_call(
        _mlp_pool_body,
        grid=(NBLK,),
        in_specs=in_specs,
        out_specs=full((G, C)),
        out_shape=jax.ShapeDtypeStruct((G, C), jnp.float32),
        scratch_shapes=[pltpu.VMEM((G, H), jnp.float32),
                        pltpu.VMEM((G, 1), jnp.float32)],
    )(*h_chunks, lap, *agg_chunks, agglap, W1, b1.reshape(1, H),
      W2, b2.reshape(1, H), ep, batchr, Wp, bp.reshape(1, C))


def _pool_body(h0, h1, h2, h3, b_ref, Wp_ref, bp_ref, out_ref, psum, cnt):
    i = pl.program_id(0)

    @pl.when(i == 0)
    def _init():
        psum[...] = jnp.zeros_like(psum)
        cnt[...] = jnp.zeros_like(cnt)

    batch = b_ref[0, 0, :]
    ids = lax.broadcasted_iota(jnp.int32, (G, RBLK), 0)
    mask = (batch[None, :] == ids).astype(jnp.float32)
    hcat = jnp.concatenate([h0[...], h1[...], h2[...], h3[...]], axis=1)
    psum[...] += jnp.dot(mask, hcat, preferred_element_type=jnp.float32)
    cnt[...] += jnp.sum(mask, axis=1, keepdims=True)

    @pl.when(i == NBLK - 1)
    def _final():
        pooled = psum[...] / jnp.maximum(cnt[...], 1.0)
        out_ref[...] = (jnp.dot(pooled, Wp_ref[...],
                                preferred_element_type=jnp.float32)
                        + bp_ref[...])


@jax.jit
def _pool(h_chunks, batch, Wp, bp):
    row_spec = pl.BlockSpec((RBLK, 128), lambda i: (i, 0))
    batchr = batch.reshape(NBLK, 1, RBLK)
    full = lambda shape: pl.BlockSpec(shape, lambda i: tuple(0 for _ in shape))
    return pl.pallas_call(
        _pool_body,
        grid=(NBLK,),
        in_specs=[row_spec] * 4 + [
            pl.BlockSpec((1, 1, RBLK), lambda i: (i, 0, 0)),
            full(Wp.shape), full((1, C))],
        out_specs=full((G, C)),
        out_shape=jax.ShapeDtypeStruct((G, C), jnp.float32),
        scratch_shapes=[pltpu.VMEM((G, H), jnp.float32),
                        pltpu.VMEM((G, 1), jnp.float32)],
    )(*h_chunks, batchr, Wp, bp.reshape(1, C))


def kernel(x, edge_index, laplace_feats, batch,
           W1_0, b1_0, W2_0, b2_0, eps_0,
           W1_1, b1_1, W2_1, b2_1, eps_1,
           W1_2, b1_2, W2_2, b2_2, eps_2,
           Wp, bp):
    srcr = edge_index[0].reshape(NW, NGROUP, 1, GROUP, STEP)
    dstr = edge_index[1].reshape(NW, NGROUP, 1, GROUP, STEP)
    sd = jnp.concatenate([srcr, dstr], axis=2)
    z128 = jnp.zeros((ROWS_A, 128), dtype=jnp.float32)

    lappad = jnp.pad(laplace_feats, ((0, 0), (0, 128 - K)))
    aggx, agglap_p = _sc_spmm_multi([x, lappad], sd, z128)
    agglap = agglap_p[:, :, :K]

    ep0 = jnp.reshape(1.0 + eps_0, (1, 1))
    h1 = _mlp([x], laplace_feats, [aggx], agglap,
              W1_0, b1_0, W2_0, b2_0, ep0, nch=1, residual=False)

    agg1 = _sc_spmm_multi(h1, sd, z128)
    ep1 = jnp.reshape(1.0 + eps_1, (1, 1))
    h2 = _mlp(h1, laplace_feats, agg1, agglap,
              W1_1, b1_1, W2_1, b2_1, ep1, nch=4, residual=True)

    agg2 = _sc_spmm_multi(h2, sd, z128)
    ep2 = jnp.reshape(1.0 + eps_2, (1, 1))
    return _mlp_pool(h2, laplace_feats, agg2, agglap,
                     W1_2, b1_2, W2_2, b2_2, ep2, batch, Wp, bp)


# speedup vs baseline: 3.6681x; 1.0050x over previous
import functools

import jax
import jax.numpy as jnp
from jax import lax
from jax.experimental import pallas as pl
from jax.experimental.pallas import tpu as pltpu
from jax.experimental.pallas import tpu_sc as plsc

N = 10000
E = 320000
D = 128
K = 16
H = 512
C = 10
G = 64

NUM_CORES = 2
NUM_SUBCORES = 16
NW = NUM_CORES * NUM_SUBCORES
EPW = E // NW
STEP = 50
NSTEP = EPW // STEP
GROUP = 40
NGROUP = NSTEP // GROUP
ROWS_A = 624
TAIL = N - NUM_SUBCORES * ROWS_A
TAIL0 = NUM_SUBCORES * ROWS_A


@jax.jit
def _sc_spmm_multi(tables, sd_idx, zeros):
    ntab = len(tables)
    mesh = plsc.VectorSubcoreMesh(core_axis_name="c", subcore_axis_name="s")
    fc = 128

    @functools.partial(
        pl.kernel,
        mesh=mesh,
        out_type=[jax.ShapeDtypeStruct((NUM_CORES, N, fc), jnp.float32)] * ntab,
        scratch_types=[
            pltpu.VMEM((2, 2, GROUP, STEP), jnp.int32),
            pltpu.VMEM((4, STEP, fc), jnp.float32),
            pltpu.VMEM_SHARED((N, fc), jnp.float32),
        ] + [pltpu.SemaphoreType.DMA] * 10,
    )
    def k(*refs):
        table_refs = refs[:ntab]
        sd_hbm, zeros_hbm = refs[ntab:ntab + 2]
        out_refs = refs[ntab + 2:2 * ntab + 2]
        (sd, rows, acc_ref, sa0, sa1, sa2, sa3, sb0, sb1, sb2, sb3,
         isem0, isem1) = refs[2 * ntab + 2:]
        semsa = (sa0, sa1, sa2, sa3)
        semsb = (sb0, sb1, sb2, sb3)
        isems = (isem0, isem1)
        c = lax.axis_index("c")
        s = lax.axis_index("s")
        wid = c * NUM_SUBCORES + s
        r0 = s * ROWS_A

        for t in range(ntab):
            table_hbm = table_refs[t]
            pltpu.async_copy(sd_hbm.at[wid, 0], sd.at[0], isem0)
            pltpu.sync_copy(zeros_hbm.at[pl.ds(0, ROWS_A)],
                            acc_ref.at[pl.ds(r0, ROWS_A)])

            @pl.when(s == NUM_SUBCORES - 1)
            def _ztail():
                pltpu.sync_copy(zeros_hbm.at[pl.ds(0, TAIL)],
                                acc_ref.at[pl.ds(TAIL0, TAIL)])

            plsc.subcore_barrier()

            for g in range(NGROUP):
                gb = g % 2
                pltpu.make_async_copy(sd_hbm.at[wid, g], sd.at[gb],
                                      isems[gb]).wait()
                if g + 1 < NGROUP:
                    pltpu.async_copy(sd_hbm.at[wid, g + 1], sd.at[1 - gb],
                                     isems[1 - gb])
                sidx = sd.at[gb, 0]
                didx = sd.at[gb, 1]
                HS = STEP // 2

                def _gather(jj, bb):
                    pltpu.async_copy(
                        table_hbm.at[sidx.at[jj, pl.ds(0, HS)]],
                        rows.at[bb, pl.ds(0, HS)], semsa[bb])
                    pltpu.async_copy(
                        table_hbm.at[sidx.at[jj, pl.ds(HS, HS)]],
                        rows.at[bb, pl.ds(HS, HS)], semsb[bb])

                for b in range(3):
                    _gather(b, b)

                def quad(q, carry2):
                    j0 = 4 * q
                    for b in range(4):
                        j = j0 + b
                        jn = j + 3
                        bn = (b + 3) % 4

                        @pl.when(jn < GROUP)
                        def _next():
                            _gather(jn, bn)

                        pltpu.make_async_copy(
                            table_hbm.at[sidx.at[j, pl.ds(0, HS)]],
                            rows.at[b, pl.ds(0, HS)], semsa[b]).wait()
                        pltpu.make_async_copy(
                            table_hbm.at[sidx.at[j, pl.ds(HS, HS)]],
                            rows.at[b, pl.ds(HS, HS)], semsb[b]).wait()
                        pltpu.sync_copy(rows.at[b], acc_ref.at[didx.at[j]],
                                        add=True)
                    return carry2

                lax.fori_loop(0, GROUP // 4, quad, 0)

            plsc.subcore_barrier()
            pltpu.sync_copy(acc_ref.at[pl.ds(r0, ROWS_A)],
                            out_refs[t].at[c, pl.ds(r0, ROWS_A)])

            @pl.when(s == NUM_SUBCORES - 1)
            def _otail():
                pltpu.sync_copy(acc_ref.at[pl.ds(TAIL0, TAIL)],
                                out_refs[t].at[c, pl.ds(TAIL0, TAIL)])

    return k(*tables, sd_idx, zeros)


RBLK = 400
NBLK = N // RBLK


def _mlp_body(nch, residual, *refs):
    i = 0
    h_refs = refs[i:i + nch]; i += nch
    lap_ref = refs[i]; i += 1
    a_refs = refs[i:i + nch]; i += nch
    alap_ref = refs[i]; i += 1
    W1_ref = refs[i]; i += 1
    b1_ref = refs[i]; i += 1
    W2_ref = refs[i]; i += 1
    b2_ref = refs[i]; i += 1
    ep_ref = refs[i]; i += 1
    o_refs = refs[i:i + 4]

    ep = ep_ref[0, 0]
    acc = jnp.zeros((RBLK, H), dtype=jnp.float32)
    for cidx in range(nch):
        a = a_refs[cidx]
        z = ep * h_refs[cidx][...] + a[0] + a[1]
        w = W1_ref[cidx * 128:(cidx + 1) * 128, :]
        acc = acc + jnp.dot(z, w, preferred_element_type=jnp.float32)
    zlap = ep * lap_ref[...] + alap_ref[0] + alap_ref[1]
    wlap = W1_ref[nch * 128:nch * 128 + K, :]
    acc = acc + jnp.dot(zlap, wlap, preferred_element_type=jnp.float32)
    t = jnp.maximum(acc + b1_ref[...], 0.0)
    o = jnp.dot(t, W2_ref[...], preferred_element_type=jnp.float32) + b2_ref[...]
    o = jnp.maximum(o, 0.0)
    for cidx in range(4):
        oc = o[:, cidx * 128:(cidx + 1) * 128]
        if residual:
            oc = oc + h_refs[cidx][...]
        o_refs[cidx][...] = oc


@functools.partial(jax.jit, static_argnames=("nch", "residual"))
def _mlp(h_chunks, lap, agg_chunks, agglap, W1, b1, W2, b2, ep,
         nch, residual):
    row_spec = pl.BlockSpec((RBLK, 128), lambda i: (i, 0))
    lap_spec = pl.BlockSpec((RBLK, K), lambda i: (i, 0))
    agg_spec = pl.BlockSpec((2, RBLK, 128), lambda i: (0, i, 0))
    alap_spec = pl.BlockSpec((2, RBLK, K), lambda i: (0, i, 0))
    full = lambda shape: pl.BlockSpec(shape, lambda i: tuple(0 for _ in shape))
    smem = pl.BlockSpec(memory_space=pltpu.SMEM)

    in_specs = ([row_spec] * nch + [lap_spec] + [agg_spec] * nch +
                [alap_spec, full(W1.shape), full((1, H)), full(W2.shape),
                 full((1, H)), smem])
    out_specs = [row_spec] * 4
    out_shape = [jax.ShapeDtypeStruct((N, 128), jnp.float32)] * 4

    return pl.pallas_call(
        functools.partial(_mlp_body, nch, residual),
        grid=(NBLK,),
        in_specs=in_specs,
        out_specs=out_specs,
        out_shape=out_shape,
    )(*h_chunks, lap, *agg_chunks, agglap, W1, b1.reshape(1, H),
      W2, b2.reshape(1, H), ep)


def _mlp_pool_body(*refs):
    h_refs = refs[0:4]
    lap_ref = refs[4]
    a_refs = refs[5:9]
    alap_ref = refs[9]
    W1_ref, b1_ref, W2_ref, b2_ref, ep_ref, b_ref, Wp_ref, bp_ref = refs[10:18]
    out_ref = refs[18]
    psum, cnt = refs[19:21]

    i = pl.program_id(0)

    @pl.when(i == 0)
    def _init():
        psum[...] = jnp.zeros_like(psum)
        cnt[...] = jnp.zeros_like(cnt)

    ep = ep_ref[0, 0]
    acc = jnp.zeros((RBLK, H), dtype=jnp.float32)
    for cidx in range(4):
        a = a_refs[cidx]
        z = (ep * h_refs[cidx][...] + a[0].astype(jnp.float32)
             + a[1].astype(jnp.float32))
        w = W1_ref[cidx * 128:(cidx + 1) * 128, :]
        acc = acc + jnp.dot(z, w, preferred_element_type=jnp.float32)
    zlap = (ep * lap_ref[...] + alap_ref[0].astype(jnp.float32)
            + alap_ref[1].astype(jnp.float32))
    wlap = W1_ref[4 * 128:4 * 128 + K, :]
    acc = acc + jnp.dot(zlap, wlap, preferred_element_type=jnp.float32)
    t = jnp.maximum(acc + b1_ref[...], 0.0)
    o = jnp.dot(t, W2_ref[...], preferred_element_type=jnp.float32) + b2_ref[...]
    o = jnp.maximum(o, 0.0)
    hcat = jnp.concatenate([h_refs[c][...] for c in range(4)], axis=1)
    h3 = o + hcat

    batch = b_ref[0, 0, :]
    ids = lax.broadcasted_iota(jnp.int32, (G, RBLK), 0)
    mask = (batch[None, :] == ids).astype(jnp.float32)
    psum[...] += jnp.dot(mask, h3, preferred_element_type=jnp.float32)
    cnt[...] += jnp.sum(mask, axis=1, keepdims=True)

    @pl.when(i == NBLK - 1)
    def _final():
        pooled = psum[...] / jnp.maximum(cnt[...], 1.0)
        out_ref[...] = (jnp.dot(pooled, Wp_ref[...],
                                preferred_element_type=jnp.float32)
                        + bp_ref[...])


@jax.jit
def _mlp_pool(h_chunks, lap, agg_chunks, agglap, W1, b1, W2, b2, ep,
              batch, Wp, bp):
    row_spec = pl.BlockSpec((RBLK, 128), lambda i: (i, 0))
    lap_spec = pl.BlockSpec((RBLK, K), lambda i: (i, 0))
    agg_spec = pl.BlockSpec((2, RBLK, 128), lambda i: (0, i, 0))
    alap_spec = pl.BlockSpec((2, RBLK, K), lambda i: (0, i, 0))
    full = lambda shape: pl.BlockSpec(shape, lambda i: tuple(0 for _ in shape))
    smem = pl.BlockSpec(memory_space=pltpu.SMEM)
    batchr = batch.reshape(NBLK, 1, RBLK)

    in_specs = ([row_spec] * 4 + [lap_spec] + [agg_spec] * 4 +
                [alap_spec, full(W1.shape), full((1, H)), full(W2.shape),
                 full((1, H)), smem,
                 pl.BlockSpec((1, 1, RBLK), lambda i: (i, 0, 0)),
                 full(Wp.shape), full((1, C))])

    return pl.pallas_call(
        _mlp_pool_body,
        grid=(NBLK,),
        in_specs=in_specs,
        out_specs=full((G, C)),
        out_shape=jax.ShapeDtypeStruct((G, C), jnp.float32),
        scratch_shapes=[pltpu.VMEM((G, H), jnp.float32),
                        pltpu.VMEM((G, 1), jnp.float32)],
    )(*h_chunks, lap, *agg_chunks, agglap, W1, b1.reshape(1, H),
      W2, b2.reshape(1, H), ep, batchr, Wp, bp.reshape(1, C))


def _pool_body(h0, h1, h2, h3, b_ref, Wp_ref, bp_ref, out_ref, psum, cnt):
    i = pl.program_id(0)

    @pl.when(i == 0)
    def _init():
        psum[...] = jnp.zeros_like(psum)
        cnt[...] = jnp.zeros_like(cnt)

    batch = b_ref[0, 0, :]
    ids = lax.broadcasted_iota(jnp.int32, (G, RBLK), 0)
    mask = (batch[None, :] == ids).astype(jnp.float32)
    hcat = jnp.concatenate([h0[...], h1[...], h2[...], h3[...]], axis=1)
    psum[...] += jnp.dot(mask, hcat, preferred_element_type=jnp.float32)
    cnt[...] += jnp.sum(mask, axis=1, keepdims=True)

    @pl.when(i == NBLK - 1)
    def _final():
        pooled = psum[...] / jnp.maximum(cnt[...], 1.0)
        out_ref[...] = (jnp.dot(pooled, Wp_ref[...],
                                preferred_element_type=jnp.float32)
                        + bp_ref[...])


@jax.jit
def _pool(h_chunks, batch, Wp, bp):
    row_spec = pl.BlockSpec((RBLK, 128), lambda i: (i, 0))
    batchr = batch.reshape(NBLK, 1, RBLK)
    full = lambda shape: pl.BlockSpec(shape, lambda i: tuple(0 for _ in shape))
    return pl.pallas_call(
        _pool_body,
        grid=(NBLK,),
        in_specs=[row_spec] * 4 + [
            pl.BlockSpec((1, 1, RBLK), lambda i: (i, 0, 0)),
            full(Wp.shape), full((1, C))],
        out_specs=full((G, C)),
        out_shape=jax.ShapeDtypeStruct((G, C), jnp.float32),
        scratch_shapes=[pltpu.VMEM((G, H), jnp.float32),
                        pltpu.VMEM((G, 1), jnp.float32)],
    )(*h_chunks, batchr, Wp, bp.reshape(1, C))


def kernel(x, edge_index, laplace_feats, batch,
           W1_0, b1_0, W2_0, b2_0, eps_0,
           W1_1, b1_1, W2_1, b2_1, eps_1,
           W1_2, b1_2, W2_2, b2_2, eps_2,
           Wp, bp):
    srcr = edge_index[0].reshape(NW, NGROUP, 1, GROUP, STEP)
    dstr = edge_index[1].reshape(NW, NGROUP, 1, GROUP, STEP)
    sd = jnp.concatenate([srcr, dstr], axis=2)
    z128 = jnp.zeros((ROWS_A, 128), dtype=jnp.float32)

    lappad = jnp.pad(laplace_feats, ((0, 0), (0, 128 - K)))
    aggx, agglap_p = _sc_spmm_multi([x, lappad], sd, z128)
    agglap = agglap_p[:, :, :K]

    ep0 = jnp.reshape(1.0 + eps_0, (1, 1))
    h1 = _mlp([x], laplace_feats, [aggx], agglap,
              W1_0, b1_0, W2_0, b2_0, ep0, nch=1, residual=False)

    agg1 = _sc_spmm_multi(h1, sd, z128)
    ep1 = jnp.reshape(1.0 + eps_1, (1, 1))
    h2 = _mlp(h1, laplace_feats, agg1, agglap,
              W1_1, b1_1, W2_1, b2_1, ep1, nch=4, residual=True)

    agg2 = _sc_spmm_multi(h2, sd, z128)
    ep2 = jnp.reshape(1.0 + eps_2, (1, 1))
    return _mlp_pool(h2, laplace_feats, agg2, agglap,
                     W1_2, b1_2, W2_2, b2_2, ep2, batch, Wp, bp)


# speedup vs baseline: 3.7966x; 1.0350x over previous
import functools

import jax
import jax.numpy as jnp
from jax import lax
from jax.experimental import pallas as pl
from jax.experimental.pallas import tpu as pltpu
from jax.experimental.pallas import tpu_sc as plsc

N = 10000
E = 320000
D = 128
K = 16
H = 512
C = 10
G = 64

NUM_CORES = 2
NUM_SUBCORES = 16
NW = NUM_CORES * NUM_SUBCORES
EPW = E // NW
STEP = 50
NSTEP = EPW // STEP
GROUP = 40
NGROUP = NSTEP // GROUP
ROWS_A = 624
TAIL = N - NUM_SUBCORES * ROWS_A
TAIL0 = NUM_SUBCORES * ROWS_A


@jax.jit
def _sc_spmm_multi(tables, sd_idx, zeros):
    ntab = len(tables)
    mesh = plsc.VectorSubcoreMesh(core_axis_name="c", subcore_axis_name="s")
    fc = 128

    @functools.partial(
        pl.kernel,
        mesh=mesh,
        out_type=[jax.ShapeDtypeStruct((NUM_CORES, N, fc), jnp.float32)] * ntab,
        scratch_types=[
            pltpu.VMEM((2, 2, GROUP, STEP), jnp.int32),
            pltpu.VMEM((4, STEP, fc), jnp.float32),
            pltpu.VMEM_SHARED((N, fc), jnp.float32),
        ] + [pltpu.SemaphoreType.DMA] * 10,
    )
    def k(*refs):
        table_refs = refs[:ntab]
        sd_hbm, zeros_hbm = refs[ntab:ntab + 2]
        out_refs = refs[ntab + 2:2 * ntab + 2]
        (sd, rows, acc_ref, sa0, sa1, sa2, sa3, sb0, sb1, sb2, sb3,
         isem0, isem1) = refs[2 * ntab + 2:]
        semsa = (sa0, sa1, sa2, sa3)
        semsb = (sb0, sb1, sb2, sb3)
        isems = (isem0, isem1)
        c = lax.axis_index("c")
        s = lax.axis_index("s")
        wid = c * NUM_SUBCORES + s
        r0 = s * ROWS_A

        for t in range(ntab):
            table_hbm = table_refs[t]
            pltpu.async_copy(sd_hbm.at[wid, 0], sd.at[0], isem0)
            pltpu.sync_copy(zeros_hbm.at[pl.ds(0, ROWS_A)],
                            acc_ref.at[pl.ds(r0, ROWS_A)])

            @pl.when(s == NUM_SUBCORES - 1)
            def _ztail():
                pltpu.sync_copy(zeros_hbm.at[pl.ds(0, TAIL)],
                                acc_ref.at[pl.ds(TAIL0, TAIL)])

            plsc.subcore_barrier()

            for g in range(NGROUP):
                gb = g % 2
                pltpu.make_async_copy(sd_hbm.at[wid, g], sd.at[gb],
                                      isems[gb]).wait()
                if g + 1 < NGROUP:
                    pltpu.async_copy(sd_hbm.at[wid, g + 1], sd.at[1 - gb],
                                     isems[1 - gb])
                sidx = sd.at[gb, 0]
                didx = sd.at[gb, 1]
                HS = STEP // 2

                def _gather(jj, bb):
                    pltpu.async_copy(
                        table_hbm.at[sidx.at[jj, pl.ds(0, HS)]],
                        rows.at[bb, pl.ds(0, HS)], semsa[bb])
                    pltpu.async_copy(
                        table_hbm.at[sidx.at[jj, pl.ds(HS, HS)]],
                        rows.at[bb, pl.ds(HS, HS)], semsb[bb])

                for b in range(3):
                    _gather(b, b)

                def quad(q, carry2):
                    j0 = 4 * q
                    for b in range(4):
                        j = j0 + b
                        jn = j + 3
                        bn = (b + 3) % 4

                        @pl.when(jn < GROUP)
                        def _next():
                            _gather(jn, bn)

                        pltpu.make_async_copy(
                            table_hbm.at[sidx.at[j, pl.ds(0, HS)]],
                            rows.at[b, pl.ds(0, HS)], semsa[b]).wait()
                        pltpu.make_async_copy(
                            table_hbm.at[sidx.at[j, pl.ds(HS, HS)]],
                            rows.at[b, pl.ds(HS, HS)], semsb[b]).wait()
                        pltpu.sync_copy(rows.at[b], acc_ref.at[didx.at[j]],
                                        add=True)
                    return carry2

                lax.fori_loop(0, GROUP // 4, quad, 0)

            plsc.subcore_barrier()
            pltpu.sync_copy(acc_ref.at[pl.ds(r0, ROWS_A)],
                            out_refs[t].at[c, pl.ds(r0, ROWS_A)])

            @pl.when(s == NUM_SUBCORES - 1)
            def _otail():
                pltpu.sync_copy(acc_ref.at[pl.ds(TAIL0, TAIL)],
                                out_refs[t].at[c, pl.ds(TAIL0, TAIL)])

    return k(*tables, sd_idx, zeros)


@jax.jit
def _sc_spmm_lap(table, sd_idx, zeros):
    mesh = plsc.VectorSubcoreMesh(core_axis_name="c", subcore_axis_name="s")
    fc = K

    @functools.partial(
        pl.kernel,
        mesh=mesh,
        out_type=jax.ShapeDtypeStruct((NUM_CORES, N, fc), jnp.float32),
        scratch_types=[
            pltpu.VMEM((2, 2, GROUP, STEP), jnp.int32),
            pltpu.VMEM((4, STEP, fc), jnp.float32),
            pltpu.VMEM_SHARED((N, fc), jnp.float32),
        ] + [pltpu.SemaphoreType.DMA] * 6,
        compiler_params=pltpu.CompilerParams(use_tc_tiling_on_sc=False),
    )
    def k(table_hbm, sd_hbm, zeros_hbm, out_hbm, sd, rows, acc_ref,
          sem0, sem1, sem2, sem3, isem0, isem1):
        sems = (sem0, sem1, sem2, sem3)
        isems = (isem0, isem1)
        c = lax.axis_index("c")
        s = lax.axis_index("s")
        wid = c * NUM_SUBCORES + s
        r0 = s * ROWS_A

        pltpu.async_copy(sd_hbm.at[wid, 0], sd.at[0], isem0)
        pltpu.sync_copy(zeros_hbm.at[pl.ds(0, ROWS_A)],
                        acc_ref.at[pl.ds(r0, ROWS_A)])

        @pl.when(s == NUM_SUBCORES - 1)
        def _ztail():
            pltpu.sync_copy(zeros_hbm.at[pl.ds(0, TAIL)],
                            acc_ref.at[pl.ds(TAIL0, TAIL)])

        plsc.subcore_barrier()

        for g in range(NGROUP):
            gb = g % 2
            pltpu.make_async_copy(sd_hbm.at[wid, g], sd.at[gb],
                                  isems[gb]).wait()
            if g + 1 < NGROUP:
                pltpu.async_copy(sd_hbm.at[wid, g + 1], sd.at[1 - gb],
                                 isems[1 - gb])
            sidx = sd.at[gb, 0]
            didx = sd.at[gb, 1]
            for b in range(3):
                pltpu.async_copy(table_hbm.at[sidx.at[b]], rows.at[b],
                                 sems[b])

            def quad(q, carry2):
                j0 = 4 * q
                for b in range(4):
                    j = j0 + b
                    jn = j + 3
                    bn = (b + 3) % 4

                    @pl.when(jn < GROUP)
                    def _next():
                        pltpu.async_copy(table_hbm.at[sidx.at[jn]],
                                         rows.at[bn], sems[bn])

                    pltpu.make_async_copy(table_hbm.at[sidx.at[j]],
                                          rows.at[b], sems[b]).wait()
                    pltpu.sync_copy(rows.at[b], acc_ref.at[didx.at[j]],
                                    add=True)
                return carry2

            lax.fori_loop(0, GROUP // 4, quad, 0)

        plsc.subcore_barrier()
        pltpu.sync_copy(acc_ref.at[pl.ds(r0, ROWS_A)],
                        out_hbm.at[c, pl.ds(r0, ROWS_A)])

        @pl.when(s == NUM_SUBCORES - 1)
        def _otail():
            pltpu.sync_copy(acc_ref.at[pl.ds(TAIL0, TAIL)],
                            out_hbm.at[c, pl.ds(TAIL0, TAIL)])

    return k(table, sd_idx, zeros)


RBLK = 400
NBLK = N // RBLK


def _mlp_body(nch, residual, *refs):
    i = 0
    h_refs = refs[i:i + nch]; i += nch
    lap_ref = refs[i]; i += 1
    a_refs = refs[i:i + nch]; i += nch
    alap_ref = refs[i]; i += 1
    W1_ref = refs[i]; i += 1
    b1_ref = refs[i]; i += 1
    W2_ref = refs[i]; i += 1
    b2_ref = refs[i]; i += 1
    ep_ref = refs[i]; i += 1
    o_refs = refs[i:i + 4]

    ep = ep_ref[0, 0]
    acc = jnp.zeros((RBLK, H), dtype=jnp.float32)
    for cidx in range(nch):
        a = a_refs[cidx]
        z = ep * h_refs[cidx][...] + a[0] + a[1]
        w = W1_ref[cidx * 128:(cidx + 1) * 128, :]
        acc = acc + jnp.dot(z, w, preferred_element_type=jnp.float32)
    zlap = ep * lap_ref[...] + alap_ref[0] + alap_ref[1]
    wlap = W1_ref[nch * 128:nch * 128 + K, :]
    acc = acc + jnp.dot(zlap, wlap, preferred_element_type=jnp.float32)
    t = jnp.maximum(acc + b1_ref[...], 0.0)
    o = jnp.dot(t, W2_ref[...], preferred_element_type=jnp.float32) + b2_ref[...]
    o = jnp.maximum(o, 0.0)
    for cidx in range(4):
        oc = o[:, cidx * 128:(cidx + 1) * 128]
        if residual:
            oc = oc + h_refs[cidx][...]
        o_refs[cidx][...] = oc


@functools.partial(jax.jit, static_argnames=("nch", "residual"))
def _mlp(h_chunks, lap, agg_chunks, agglap, W1, b1, W2, b2, ep,
         nch, residual):
    row_spec = pl.BlockSpec((RBLK, 128), lambda i: (i, 0))
    lap_spec = pl.BlockSpec((RBLK, K), lambda i: (i, 0))
    agg_spec = pl.BlockSpec((2, RBLK, 128), lambda i: (0, i, 0))
    alap_spec = pl.BlockSpec((2, RBLK, K), lambda i: (0, i, 0))
    full = lambda shape: pl.BlockSpec(shape, lambda i: tuple(0 for _ in shape))
    smem = pl.BlockSpec(memory_space=pltpu.SMEM)

    in_specs = ([row_spec] * nch + [lap_spec] + [agg_spec] * nch +
                [alap_spec, full(W1.shape), full((1, H)), full(W2.shape),
                 full((1, H)), smem])
    out_specs = [row_spec] * 4
    out_shape = [jax.ShapeDtypeStruct((N, 128), jnp.float32)] * 4

    return pl.pallas_call(
        functools.partial(_mlp_body, nch, residual),
        grid=(NBLK,),
        in_specs=in_specs,
        out_specs=out_specs,
        out_shape=out_shape,
    )(*h_chunks, lap, *agg_chunks, agglap, W1, b1.reshape(1, H),
      W2, b2.reshape(1, H), ep)


def _mlp_pool_body(*refs):
    h_refs = refs[0:4]
    lap_ref = refs[4]
    a_refs = refs[5:9]
    alap_ref = refs[9]
    W1_ref, b1_ref, W2_ref, b2_ref, ep_ref, b_ref, Wp_ref, bp_ref = refs[10:18]
    out_ref = refs[18]
    psum, cnt = refs[19:21]

    i = pl.program_id(0)

    @pl.when(i == 0)
    def _init():
        psum[...] = jnp.zeros_like(psum)
        cnt[...] = jnp.zeros_like(cnt)

    ep = ep_ref[0, 0]
    acc = jnp.zeros((RBLK, H), dtype=jnp.float32)
    for cidx in range(4):
        a = a_refs[cidx]
        z = (ep * h_refs[cidx][...] + a[0].astype(jnp.float32)
             + a[1].astype(jnp.float32))
        w = W1_ref[cidx * 128:(cidx + 1) * 128, :]
        acc = acc + jnp.dot(z, w, preferred_element_type=jnp.float32)
    zlap = (ep * lap_ref[...] + alap_ref[0].astype(jnp.float32)
            + alap_ref[1].astype(jnp.float32))
    wlap = W1_ref[4 * 128:4 * 128 + K, :]
    acc = acc + jnp.dot(zlap, wlap, preferred_element_type=jnp.float32)
    t = jnp.maximum(acc + b1_ref[...], 0.0)
    o = jnp.dot(t, W2_ref[...], preferred_element_type=jnp.float32) + b2_ref[...]
    o = jnp.maximum(o, 0.0)
    hcat = jnp.concatenate([h_refs[c][...] for c in range(4)], axis=1)
    h3 = o + hcat

    batch = b_ref[0, 0, :]
    ids = lax.broadcasted_iota(jnp.int32, (G, RBLK), 0)
    mask = (batch[None, :] == ids).astype(jnp.float32)
    psum[...] += jnp.dot(mask, h3, preferred_element_type=jnp.float32)
    cnt[...] += jnp.sum(mask, axis=1, keepdims=True)

    @pl.when(i == NBLK - 1)
    def _final():
        pooled = psum[...] / jnp.maximum(cnt[...], 1.0)
        out_ref[...] = (jnp.dot(pooled, Wp_ref[...],
                                preferred_element_type=jnp.float32)
                        + bp_ref[...])


@jax.jit
def _mlp_pool(h_chunks, lap, agg_chunks, agglap, W1, b1, W2, b2, ep,
              batch, Wp, bp):
    row_spec = pl.BlockSpec((RBLK, 128), lambda i: (i, 0))
    lap_spec = pl.BlockSpec((RBLK, K), lambda i: (i, 0))
    agg_spec = pl.BlockSpec((2, RBLK, 128), lambda i: (0, i, 0))
    alap_spec = pl.BlockSpec((2, RBLK, K), lambda i: (0, i, 0))
    full = lambda shape: pl.BlockSpec(shape, lambda i: tuple(0 for _ in shape))
    smem = pl.BlockSpec(memory_space=pltpu.SMEM)
    batchr = batch.reshape(NBLK, 1, RBLK)

    in_specs = ([row_spec] * 4 + [lap_spec] + [agg_spec] * 4 +
                [alap_spec, full(W1.shape), full((1, H)), full(W2.shape),
                 full((1, H)), smem,
                 pl.BlockSpec((1, 1, RBLK), lambda i: (i, 0, 0)),
                 full(Wp.shape), full((1, C))])

    return pl.pallas_call(
        _mlp_pool_body,
        grid=(NBLK,),
        in_specs=in_specs,
        out_specs=full((G, C)),
        out_shape=jax.ShapeDtypeStruct((G, C), jnp.float32),
        scratch_shapes=[pltpu.VMEM((G, H), jnp.float32),
                        pltpu.VMEM((G, 1), jnp.float32)],
    )(*h_chunks, lap, *agg_chunks, agglap, W1, b1.reshape(1, H),
      W2, b2.reshape(1, H), ep, batchr, Wp, bp.reshape(1, C))


def _pool_body(h0, h1, h2, h3, b_ref, Wp_ref, bp_ref, out_ref, psum, cnt):
    i = pl.program_id(0)

    @pl.when(i == 0)
    def _init():
        psum[...] = jnp.zeros_like(psum)
        cnt[...] = jnp.zeros_like(cnt)

    batch = b_ref[0, 0, :]
    ids = lax.broadcasted_iota(jnp.int32, (G, RBLK), 0)
    mask = (batch[None, :] == ids).astype(jnp.float32)
    hcat = jnp.concatenate([h0[...], h1[...], h2[...], h3[...]], axis=1)
    psum[...] += jnp.dot(mask, hcat, preferred_element_type=jnp.float32)
    cnt[...] += jnp.sum(mask, axis=1, keepdims=True)

    @pl.when(i == NBLK - 1)
    def _final():
        pooled = psum[...] / jnp.maximum(cnt[...], 1.0)
        out_ref[...] = (jnp.dot(pooled, Wp_ref[...],
                                preferred_element_type=jnp.float32)
                        + bp_ref[...])


@jax.jit
def _pool(h_chunks, batch, Wp, bp):
    row_spec = pl.BlockSpec((RBLK, 128), lambda i: (i, 0))
    batchr = batch.reshape(NBLK, 1, RBLK)
    full = lambda shape: pl.BlockSpec(shape, lambda i: tuple(0 for _ in shape))
    return pl.pallas_call(
        _pool_body,
        grid=(NBLK,),
        in_specs=[row_spec] * 4 + [
            pl.BlockSpec((1, 1, RBLK), lambda i: (i, 0, 0)),
            full(Wp.shape), full((1, C))],
        out_specs=full((G, C)),
        out_shape=jax.ShapeDtypeStruct((G, C), jnp.float32),
        scratch_shapes=[pltpu.VMEM((G, H), jnp.float32),
                        pltpu.VMEM((G, 1), jnp.float32)],
    )(*h_chunks, batchr, Wp, bp.reshape(1, C))


def kernel(x, edge_index, laplace_feats, batch,
           W1_0, b1_0, W2_0, b2_0, eps_0,
           W1_1, b1_1, W2_1, b2_1, eps_1,
           W1_2, b1_2, W2_2, b2_2, eps_2,
           Wp, bp):
    srcr = edge_index[0].reshape(NW, NGROUP, 1, GROUP, STEP)
    dstr = edge_index[1].reshape(NW, NGROUP, 1, GROUP, STEP)
    sd = jnp.concatenate([srcr, dstr], axis=2)
    z128 = jnp.zeros((ROWS_A, 128), dtype=jnp.float32)

    z16 = jnp.zeros((ROWS_A, K), dtype=jnp.float32)
    agglap = _sc_spmm_lap(laplace_feats, sd, z16)
    (aggx,) = _sc_spmm_multi([x], sd, z128)

    ep0 = jnp.reshape(1.0 + eps_0, (1, 1))
    h1 = _mlp([x], laplace_feats, [aggx], agglap,
              W1_0, b1_0, W2_0, b2_0, ep0, nch=1, residual=False)

    agg1 = _sc_spmm_multi(h1, sd, z128)
    ep1 = jnp.reshape(1.0 + eps_1, (1, 1))
    h2 = _mlp(h1, laplace_feats, agg1, agglap,
              W1_1, b1_1, W2_1, b2_1, ep1, nch=4, residual=True)

    agg2 = _sc_spmm_multi(h2, sd, z128)
    ep2 = jnp.reshape(1.0 + eps_2, (1, 1))
    return _mlp_pool(h2, laplace_feats, agg2, agglap,
                     W1_2, b1_2, W2_2, b2_2, ep2, batch, Wp, bp)


# speedup vs baseline: 3.8852x; 1.0234x over previous
import functools

import jax
import jax.numpy as jnp
from jax import lax
from jax.experimental import pallas as pl
from jax.experimental.pallas import tpu as pltpu
from jax.experimental.pallas import tpu_sc as plsc

N = 10000
E = 320000
D = 128
K = 16
H = 512
C = 10
G = 64

NUM_CORES = 2
NUM_SUBCORES = 16
NW = NUM_CORES * NUM_SUBCORES
EPW = E // NW
STEP = 50
NSTEP = EPW // STEP
GROUP = 40
NGROUP = NSTEP // GROUP
ROWS_A = 624
TAIL = N - NUM_SUBCORES * ROWS_A
TAIL0 = NUM_SUBCORES * ROWS_A


@jax.jit
def _sc_spmm_multi(tables, sd_idx, zeros):
    ntab = len(tables)
    mesh = plsc.VectorSubcoreMesh(core_axis_name="c", subcore_axis_name="s")
    fc = 128

    @functools.partial(
        pl.kernel,
        mesh=mesh,
        out_type=[jax.ShapeDtypeStruct((NUM_CORES, N, fc), jnp.float32)] * ntab,
        scratch_types=[
            pltpu.VMEM((2, 2, GROUP, STEP), jnp.int32),
            pltpu.VMEM((4, STEP, fc), jnp.float32),
            pltpu.VMEM_SHARED((N, fc), jnp.float32),
        ] + [pltpu.SemaphoreType.DMA] * 10,
    )
    def k(*refs):
        table_refs = refs[:ntab]
        sd_hbm, zeros_hbm = refs[ntab:ntab + 2]
        out_refs = refs[ntab + 2:2 * ntab + 2]
        (sd, rows, acc_ref, sa0, sa1, sa2, sa3, sb0, sb1, sb2, sb3,
         isem0, isem1) = refs[2 * ntab + 2:]
        semsa = (sa0, sa1, sa2, sa3)
        semsb = (sb0, sb1, sb2, sb3)
        isems = (isem0, isem1)
        c = lax.axis_index("c")
        s = lax.axis_index("s")
        wid = c * NUM_SUBCORES + s
        r0 = s * ROWS_A

        for t in range(ntab):
            table_hbm = table_refs[t]
            pltpu.async_copy(sd_hbm.at[wid, 0], sd.at[0], isem0)
            pltpu.sync_copy(zeros_hbm.at[pl.ds(0, ROWS_A)],
                            acc_ref.at[pl.ds(r0, ROWS_A)])

            @pl.when(s == NUM_SUBCORES - 1)
            def _ztail():
                pltpu.sync_copy(zeros_hbm.at[pl.ds(0, TAIL)],
                                acc_ref.at[pl.ds(TAIL0, TAIL)])

            plsc.subcore_barrier()

            for g in range(NGROUP):
                gb = g % 2
                pltpu.make_async_copy(sd_hbm.at[wid, g], sd.at[gb],
                                      isems[gb]).wait()
                if g + 1 < NGROUP:
                    pltpu.async_copy(sd_hbm.at[wid, g + 1], sd.at[1 - gb],
                                     isems[1 - gb])
                sidx = sd.at[gb, 0]
                didx = sd.at[gb, 1]
                HS = STEP // 2

                def _gather(jj, bb):
                    pltpu.async_copy(
                        table_hbm.at[sidx.at[jj, pl.ds(0, HS)]],
                        rows.at[bb, pl.ds(0, HS)], semsa[bb])
                    pltpu.async_copy(
                        table_hbm.at[sidx.at[jj, pl.ds(HS, HS)]],
                        rows.at[bb, pl.ds(HS, HS)], semsb[bb])

                for b in range(3):
                    _gather(b, b)

                def quad(q, carry2):
                    j0 = 4 * q
                    for b in range(4):
                        j = j0 + b
                        jn = j + 3
                        bn = (b + 3) % 4

                        @pl.when(jn < GROUP)
                        def _next():
                            _gather(jn, bn)

                        pltpu.make_async_copy(
                            table_hbm.at[sidx.at[j, pl.ds(0, HS)]],
                            rows.at[b, pl.ds(0, HS)], semsa[b]).wait()
                        pltpu.make_async_copy(
                            table_hbm.at[sidx.at[j, pl.ds(HS, HS)]],
                            rows.at[b, pl.ds(HS, HS)], semsb[b]).wait()
                        pltpu.sync_copy(rows.at[b], acc_ref.at[didx.at[j]],
                                        add=True)
                    return carry2

                lax.fori_loop(0, GROUP // 4, quad, 0)

            plsc.subcore_barrier()
            pltpu.sync_copy(acc_ref.at[pl.ds(r0, ROWS_A)],
                            out_refs[t].at[c, pl.ds(r0, ROWS_A)])

            @pl.when(s == NUM_SUBCORES - 1)
            def _otail():
                pltpu.sync_copy(acc_ref.at[pl.ds(TAIL0, TAIL)],
                                out_refs[t].at[c, pl.ds(TAIL0, TAIL)])

    return k(*tables, sd_idx, zeros)


@jax.jit
def _sc_spmm_lap(table, sd_idx, zeros):
    mesh = plsc.VectorSubcoreMesh(core_axis_name="c", subcore_axis_name="s")
    fc = K

    @functools.partial(
        pl.kernel,
        mesh=mesh,
        out_type=jax.ShapeDtypeStruct((NUM_CORES, N, fc), jnp.float32),
        scratch_types=[
            pltpu.VMEM((2, 2, GROUP, STEP), jnp.int32),
            pltpu.VMEM((4, STEP, fc), jnp.float32),
            pltpu.VMEM_SHARED((N, fc), jnp.float32),
        ] + [pltpu.SemaphoreType.DMA] * 6,
        compiler_params=pltpu.CompilerParams(use_tc_tiling_on_sc=False),
    )
    def k(table_hbm, sd_hbm, zeros_hbm, out_hbm, sd, rows, acc_ref,
          sem0, sem1, sem2, sem3, isem0, isem1):
        sems = (sem0, sem1, sem2, sem3)
        isems = (isem0, isem1)
        c = lax.axis_index("c")
        s = lax.axis_index("s")
        wid = c * NUM_SUBCORES + s
        r0 = s * ROWS_A

        pltpu.async_copy(sd_hbm.at[wid, 0], sd.at[0], isem0)
        pltpu.sync_copy(zeros_hbm.at[pl.ds(0, ROWS_A)],
                        acc_ref.at[pl.ds(r0, ROWS_A)])

        @pl.when(s == NUM_SUBCORES - 1)
        def _ztail():
            pltpu.sync_copy(zeros_hbm.at[pl.ds(0, TAIL)],
                            acc_ref.at[pl.ds(TAIL0, TAIL)])

        plsc.subcore_barrier()

        for g in range(NGROUP):
            gb = g % 2
            pltpu.make_async_copy(sd_hbm.at[wid, g], sd.at[gb],
                                  isems[gb]).wait()
            if g + 1 < NGROUP:
                pltpu.async_copy(sd_hbm.at[wid, g + 1], sd.at[1 - gb],
                                 isems[1 - gb])
            sidx = sd.at[gb, 0]
            didx = sd.at[gb, 1]
            for b in range(3):
                pltpu.async_copy(table_hbm.at[sidx.at[b]], rows.at[b],
                                 sems[b])

            def quad(q, carry2):
                j0 = 4 * q
                for b in range(4):
                    j = j0 + b
                    jn = j + 3
                    bn = (b + 3) % 4

                    @pl.when(jn < GROUP)
                    def _next():
                        pltpu.async_copy(table_hbm.at[sidx.at[jn]],
                                         rows.at[bn], sems[bn])

                    pltpu.make_async_copy(table_hbm.at[sidx.at[j]],
                                          rows.at[b], sems[b]).wait()
                    pltpu.sync_copy(rows.at[b], acc_ref.at[didx.at[j]],
                                    add=True)
                return carry2

            lax.fori_loop(0, GROUP // 4, quad, 0)

        plsc.subcore_barrier()
        pltpu.sync_copy(acc_ref.at[pl.ds(r0, ROWS_A)],
                        out_hbm.at[c, pl.ds(r0, ROWS_A)])

        @pl.when(s == NUM_SUBCORES - 1)
        def _otail():
            pltpu.sync_copy(acc_ref.at[pl.ds(TAIL0, TAIL)],
                            out_hbm.at[c, pl.ds(TAIL0, TAIL)])

    return k(table, sd_idx, zeros)


RBLK = 2000
NBLK = N // RBLK


def _mlp_body(nch, residual, *refs):
    i = 0
    h_refs = refs[i:i + nch]; i += nch
    lap_ref = refs[i]; i += 1
    a_refs = refs[i:i + nch]; i += nch
    alap_ref = refs[i]; i += 1
    W1_ref = refs[i]; i += 1
    b1_ref = refs[i]; i += 1
    W2_ref = refs[i]; i += 1
    b2_ref = refs[i]; i += 1
    ep_ref = refs[i]; i += 1
    o_refs = refs[i:i + 4]

    ep = ep_ref[0, 0]
    acc = jnp.zeros((RBLK, H), dtype=jnp.float32)
    for cidx in range(nch):
        a = a_refs[cidx]
        z = ep * h_refs[cidx][...] + a[0] + a[1]
        w = W1_ref[cidx * 128:(cidx + 1) * 128, :]
        acc = acc + jnp.dot(z, w, preferred_element_type=jnp.float32)
    zlap = ep * lap_ref[...] + alap_ref[0] + alap_ref[1]
    wlap = W1_ref[nch * 128:nch * 128 + K, :]
    acc = acc + jnp.dot(zlap, wlap, preferred_element_type=jnp.float32)
    t = jnp.maximum(acc + b1_ref[...], 0.0)
    o = jnp.dot(t, W2_ref[...], preferred_element_type=jnp.float32) + b2_ref[...]
    o = jnp.maximum(o, 0.0)
    for cidx in range(4):
        oc = o[:, cidx * 128:(cidx + 1) * 128]
        if residual:
            oc = oc + h_refs[cidx][...]
        o_refs[cidx][...] = oc


@functools.partial(jax.jit, static_argnames=("nch", "residual"))
def _mlp(h_chunks, lap, agg_chunks, agglap, W1, b1, W2, b2, ep,
         nch, residual):
    row_spec = pl.BlockSpec((RBLK, 128), lambda i: (i, 0))
    lap_spec = pl.BlockSpec((RBLK, K), lambda i: (i, 0))
    agg_spec = pl.BlockSpec((2, RBLK, 128), lambda i: (0, i, 0))
    alap_spec = pl.BlockSpec((2, RBLK, K), lambda i: (0, i, 0))
    full = lambda shape: pl.BlockSpec(shape, lambda i: tuple(0 for _ in shape))
    smem = pl.BlockSpec(memory_space=pltpu.SMEM)

    in_specs = ([row_spec] * nch + [lap_spec] + [agg_spec] * nch +
                [alap_spec, full(W1.shape), full((1, H)), full(W2.shape),
                 full((1, H)), smem])
    out_specs = [row_spec] * 4
    out_shape = [jax.ShapeDtypeStruct((N, 128), jnp.float32)] * 4

    return pl.pallas_call(
        functools.partial(_mlp_body, nch, residual),
        grid=(NBLK,),
        in_specs=in_specs,
        out_specs=out_specs,
        out_shape=out_shape,
    )(*h_chunks, lap, *agg_chunks, agglap, W1, b1.reshape(1, H),
      W2, b2.reshape(1, H), ep)


def _mlp_pool_body(*refs):
    h_refs = refs[0:4]
    lap_ref = refs[4]
    a_refs = refs[5:9]
    alap_ref = refs[9]
    W1_ref, b1_ref, W2_ref, b2_ref, ep_ref, b_ref, Wp_ref, bp_ref = refs[10:18]
    out_ref = refs[18]
    psum, cnt = refs[19:21]

    i = pl.program_id(0)

    @pl.when(i == 0)
    def _init():
        psum[...] = jnp.zeros_like(psum)
        cnt[...] = jnp.zeros_like(cnt)

    ep = ep_ref[0, 0]
    acc = jnp.zeros((RBLK, H), dtype=jnp.float32)
    for cidx in range(4):
        a = a_refs[cidx]
        z = (ep * h_refs[cidx][...] + a[0].astype(jnp.float32)
             + a[1].astype(jnp.float32))
        w = W1_ref[cidx * 128:(cidx + 1) * 128, :]
        acc = acc + jnp.dot(z, w, preferred_element_type=jnp.float32)
    zlap = (ep * lap_ref[...] + alap_ref[0].astype(jnp.float32)
            + alap_ref[1].astype(jnp.float32))
    wlap = W1_ref[4 * 128:4 * 128 + K, :]
    acc = acc + jnp.dot(zlap, wlap, preferred_element_type=jnp.float32)
    t = jnp.maximum(acc + b1_ref[...], 0.0)
    o = jnp.dot(t, W2_ref[...], preferred_element_type=jnp.float32) + b2_ref[...]
    o = jnp.maximum(o, 0.0)
    hcat = jnp.concatenate([h_refs[c][...] for c in range(4)], axis=1)
    h3 = o + hcat

    batch = b_ref[0, 0, :]
    ids = lax.broadcasted_iota(jnp.int32, (G, RBLK), 0)
    mask = (batch[None, :] == ids).astype(jnp.float32)
    psum[...] += jnp.dot(mask, h3, preferred_element_type=jnp.float32)
    cnt[...] += jnp.sum(mask, axis=1, keepdims=True)

    @pl.when(i == NBLK - 1)
    def _final():
        pooled = psum[...] / jnp.maximum(cnt[...], 1.0)
        out_ref[...] = (jnp.dot(pooled, Wp_ref[...],
                                preferred_element_type=jnp.float32)
                        + bp_ref[...])


@jax.jit
def _mlp_pool(h_chunks, lap, agg_chunks, agglap, W1, b1, W2, b2, ep,
              batch, Wp, bp):
    row_spec = pl.BlockSpec((RBLK, 128), lambda i: (i, 0))
    lap_spec = pl.BlockSpec((RBLK, K), lambda i: (i, 0))
    agg_spec = pl.BlockSpec((2, RBLK, 128), lambda i: (0, i, 0))
    alap_spec = pl.BlockSpec((2, RBLK, K), lambda i: (0, i, 0))
    full = lambda shape: pl.BlockSpec(shape, lambda i: tuple(0 for _ in shape))
    smem = pl.BlockSpec(memory_space=pltpu.SMEM)
    batchr = batch.reshape(NBLK, 1, RBLK)

    in_specs = ([row_spec] * 4 + [lap_spec] + [agg_spec] * 4 +
                [alap_spec, full(W1.shape), full((1, H)), full(W2.shape),
                 full((1, H)), smem,
                 pl.BlockSpec((1, 1, RBLK), lambda i: (i, 0, 0)),
                 full(Wp.shape), full((1, C))])

    return pl.pallas_call(
        _mlp_pool_body,
        grid=(NBLK,),
        in_specs=in_specs,
        out_specs=full((G, C)),
        out_shape=jax.ShapeDtypeStruct((G, C), jnp.float32),
        scratch_shapes=[pltpu.VMEM((G, H), jnp.float32),
                        pltpu.VMEM((G, 1), jnp.float32)],
    )(*h_chunks, lap, *agg_chunks, agglap, W1, b1.reshape(1, H),
      W2, b2.reshape(1, H), ep, batchr, Wp, bp.reshape(1, C))


def _pool_body(h0, h1, h2, h3, b_ref, Wp_ref, bp_ref, out_ref, psum, cnt):
    i = pl.program_id(0)

    @pl.when(i == 0)
    def _init():
        psum[...] = jnp.zeros_like(psum)
        cnt[...] = jnp.zeros_like(cnt)

    batch = b_ref[0, 0, :]
    ids = lax.broadcasted_iota(jnp.int32, (G, RBLK), 0)
    mask = (batch[None, :] == ids).astype(jnp.float32)
    hcat = jnp.concatenate([h0[...], h1[...], h2[...], h3[...]], axis=1)
    psum[...] += jnp.dot(mask, hcat, preferred_element_type=jnp.float32)
    cnt[...] += jnp.sum(mask, axis=1, keepdims=True)

    @pl.when(i == NBLK - 1)
    def _final():
        pooled = psum[...] / jnp.maximum(cnt[...], 1.0)
        out_ref[...] = (jnp.dot(pooled, Wp_ref[...],
                                preferred_element_type=jnp.float32)
                        + bp_ref[...])


@jax.jit
def _pool(h_chunks, batch, Wp, bp):
    row_spec = pl.BlockSpec((RBLK, 128), lambda i: (i, 0))
    batchr = batch.reshape(NBLK, 1, RBLK)
    full = lambda shape: pl.BlockSpec(shape, lambda i: tuple(0 for _ in shape))
    return pl.pallas_call(
        _pool_body,
        grid=(NBLK,),
        in_specs=[row_spec] * 4 + [
            pl.BlockSpec((1, 1, RBLK), lambda i: (i, 0, 0)),
            full(Wp.shape), full((1, C))],
        out_specs=full((G, C)),
        out_shape=jax.ShapeDtypeStruct((G, C), jnp.float32),
        scratch_shapes=[pltpu.VMEM((G, H), jnp.float32),
                        pltpu.VMEM((G, 1), jnp.float32)],
    )(*h_chunks, batchr, Wp, bp.reshape(1, C))


def kernel(x, edge_index, laplace_feats, batch,
           W1_0, b1_0, W2_0, b2_0, eps_0,
           W1_1, b1_1, W2_1, b2_1, eps_1,
           W1_2, b1_2, W2_2, b2_2, eps_2,
           Wp, bp):
    srcr = edge_index[0].reshape(NW, NGROUP, 1, GROUP, STEP)
    dstr = edge_index[1].reshape(NW, NGROUP, 1, GROUP, STEP)
    sd = jnp.concatenate([srcr, dstr], axis=2)
    z128 = jnp.zeros((ROWS_A, 128), dtype=jnp.float32)

    z16 = jnp.zeros((ROWS_A, K), dtype=jnp.float32)
    agglap = _sc_spmm_lap(laplace_feats, sd, z16)
    (aggx,) = _sc_spmm_multi([x], sd, z128)

    ep0 = jnp.reshape(1.0 + eps_0, (1, 1))
    h1 = _mlp([x], laplace_feats, [aggx], agglap,
              W1_0, b1_0, W2_0, b2_0, ep0, nch=1, residual=False)

    agg1 = _sc_spmm_multi(h1, sd, z128)
    ep1 = jnp.reshape(1.0 + eps_1, (1, 1))
    h2 = _mlp(h1, laplace_feats, agg1, agglap,
              W1_1, b1_1, W2_1, b2_1, ep1, nch=4, residual=True)

    agg2 = _sc_spmm_multi(h2, sd, z128)
    ep2 = jnp.reshape(1.0 + eps_2, (1, 1))
    return _mlp_pool(h2, laplace_feats, agg2, agglap,
                     W1_2, b1_2, W2_2, b2_2, ep2, batch, Wp, bp)
